# Initial kernel scaffold; baseline (speedup 1.0000x reference)
import jax
import jax.numpy as jnp
from jax.experimental import pallas as pl


def kernel(x, edge_index, edge_attr, edge_index3, recent_velocity, unit_x, unit_y, down_direction, approach_speed, inv_edge_distance3, edge_direction3, norm_inv_distance_to_boundary, norm_distance_to_boundary, acceleration_scale, velocity_scale, params):
    raise NotImplementedError("write your pallas kernel here")


# speedup vs baseline: 1.4498x; 1.4498x over previous
import functools
import jax
import jax.numpy as jnp
from jax.experimental import pallas as pl
from jax.experimental.pallas import tpu as pltpu

_EPS = 1e-5


def _dot(a, b):
    return jnp.dot(a, b, preferred_element_type=jnp.float32)


def _antisym_tail(t, b1, W2, b2, W3, b3, g):
    m = t.shape[0]
    tt = jnp.concatenate([t, -t], axis=0)
    a = jnp.maximum(tt + b1, 0.0)
    a = jnp.maximum(_dot(a, W2) + b2, 0.0)
    a = _dot(a, W3) + b3
    if g is not None:
        mu = jnp.mean(a, axis=-1, keepdims=True)
        var = jnp.mean((a - mu) ** 2, axis=-1, keepdims=True)
        a = (a - mu) * jax.lax.rsqrt(var + _EPS) * g
    return 0.5 * (a[:m] - a[m:])


def _edge_stage1_body(ea_ref,
                      eW1, eb1, eW2, eb2, eW3, eb3, eg,
                      C1, fb1, fW2, fb2, fW3, fb3, fg,
                      e1_ref, ef1_ref):
    x = ea_ref[...]
    t = _dot(x, eW1[...])
    ef = _antisym_tail(t, eb1[...], eW2[...], eb2[...], eW3[...], eb3[...], eg[...])
    t2 = _dot(ef, C1[...])
    e1 = _antisym_tail(t2, fb1[...], fW2[...], fb2[...], fW3[...], fb3[...], fg[...])
    e1_ref[...] = e1
    ef1_ref[...] = ef + e1


def _edge_stage2_body(d_ref, ef1_ref,
                      D2, C2, fb1, fW2, fb2, fW3, fb3, fg,
                      e2_ref):
    t = _dot(d_ref[...], D2[...]) + _dot(ef1_ref[...], C2[...])
    e2_ref[...] = _antisym_tail(t, fb1[...], fW2[...], fb2[...], fW3[...], fb3[...], fg[...])


def _node_stage1_body(aggr_ref, Q1, nb1, nW2, nb2, nW3, nb3, ng, nf1_ref):
    t = _dot(aggr_ref[...], Q1[...])
    nf1_ref[...] = _antisym_tail(t, nb1[...], nW2[...], nb2[...], nW3[...], nb3[...], ng[...])


def _node_stage2_body(nf1_ref, aggr_ref, ob_ref, rv_ref, nidb_ref, ndb_ref, svec_ref,
                      P2, Q2, nb1, nW2, nb2, nW3, nb3, ng,
                      oW1, ob1, oW2,
                      out_ref):
    nf1 = nf1_ref[...]
    t = _dot(nf1, P2[...]) + _dot(aggr_ref[...], Q2[...])
    nf2 = nf1 + _antisym_tail(t, nb1[...], nW2[...], nb2[...], nW3[...], nb3[...], ng[...])
    t3 = _dot(nf2, oW1[...])
    swarm = 0.5 * _dot(jnp.maximum(t3 + ob1[...], 0.0) - jnp.maximum(-t3 + ob1[...], 0.0),
                       oW2[...])
    sv = svec_ref[...]
    gdown = sv[:, 0:2]
    scale = sv[:, 2:3]
    ux = sv[:, 3:5]
    uy = sv[:, 5:7]
    out = gdown + swarm
    rv = rv_ref[...]
    ob_raw = ob_ref[...]
    cap = jnp.abs(rv * scale)
    out = out - jnp.sign(ob_raw) * jnp.minimum(jnp.abs(ob_raw), cap)
    nidb = nidb_ref[...]
    ndb = ndb_ref[...]
    wall = jnp.zeros_like(out)
    for axis, u in ((0, ux), (1, uy)):
        col = slice(axis, axis + 1)
        active = nidb[:, col] > 1e-07
        neg = ndb[:, col] < 0
        v = jnp.clip(-rv[:, col], 0.0, 1000.0)
        contrib_neg = v * u
        contrib_pos = nidb[:, col] * v * u
        wall = wall + jnp.where(active & neg, contrib_neg, 0.0)
        wall = wall + jnp.where(active & (~neg), contrib_pos, 0.0)
    out_ref[...] = out - wall


def _full(shape):
    return pl.BlockSpec(shape, lambda *_: tuple(0 for _ in shape))


def kernel(x, edge_index, edge_attr, edge_index3, recent_velocity, unit_x,
           unit_y, down_direction, approach_speed, inv_edge_distance3,
           edge_direction3, norm_inv_distance_to_boundary,
           norm_distance_to_boundary, acceleration_scale, velocity_scale,
           params):
    N = x.shape[0]
    E = edge_attr.shape[0]
    HID = 32

    pe_in = params['edge_in1']
    l1, l2 = params['layers1']
    pout = params['node_out1']

    def r(b):
        return b.reshape(1, -1)

    ein = (pe_in['Ws'][0], r(pe_in['bs'][0]), pe_in['Ws'][1], r(pe_in['bs'][1]),
           pe_in['Ws'][2], r(pe_in['bs'][2]), r(pe_in['g']))

    def edge_layer_w(p):
        W = p['edge']['Ws'][0]
        D = W[:HID] - W[HID:2 * HID]
        C = W[2 * HID:]
        tail = (r(p['edge']['bs'][0]), p['edge']['Ws'][1], r(p['edge']['bs'][1]),
                p['edge']['Ws'][2], r(p['edge']['bs'][2]), r(p['edge']['g']))
        return D, C, tail

    def node_layer_w(p):
        W = p['node']['Ws'][0]
        P = W[:HID]
        Q = W[HID:]
        tail = (r(p['node']['bs'][0]), p['node']['Ws'][1], r(p['node']['bs'][1]),
                p['node']['Ws'][2], r(p['node']['bs'][2]), r(p['node']['g']))
        return P, Q, tail

    _, C1, etail1 = edge_layer_w(l1)
    D2, C2, etail2 = edge_layer_w(l2)
    _, Q1, ntail1 = node_layer_w(l1)
    P2, Q2, ntail2 = node_layer_w(l2)

    g_const = 5.5339e-05 / acceleration_scale
    scale = velocity_scale / acceleration_scale * 60.0 / 94.0
    svec = jnp.concatenate([
        (g_const * down_direction).reshape(2),
        scale.reshape(1),
        (-unit_x * scale).reshape(2),
        (-unit_y * scale).reshape(2),
        jnp.zeros((1,), jnp.float32),
    ]).reshape(1, 8)

    src, dst = edge_index[0], edge_index[1]

    EB = 4000
    grid = (E // EB,)
    wspecs1 = [_full(w.shape) for w in ein] + [_full(C1.shape)] + \
              [_full(w.shape) for w in etail1]
    e1, ef1 = pl.pallas_call(
        _edge_stage1_body,
        grid=grid,
        in_specs=[pl.BlockSpec((EB, 14), lambda i: (i, 0))] + wspecs1,
        out_specs=[pl.BlockSpec((EB, HID), lambda i: (i, 0))] * 2,
        out_shape=[jax.ShapeDtypeStruct((E, HID), jnp.float32)] * 2,
    )(edge_attr, *ein, C1, *etail1)

    aggr1 = jax.ops.segment_sum(e1, dst, num_segments=N)

    NB = 2000
    ngrid = (N // NB,)
    wspecsn1 = [_full(Q1.shape)] + [_full(w.shape) for w in ntail1]
    nf1 = pl.pallas_call(
        _node_stage1_body,
        grid=ngrid,
        in_specs=[pl.BlockSpec((NB, HID), lambda i: (i, 0))] + wspecsn1,
        out_specs=pl.BlockSpec((NB, HID), lambda i: (i, 0)),
        out_shape=jax.ShapeDtypeStruct((N, HID), jnp.float32),
    )(aggr1, Q1, *ntail1)

    d = nf1[dst] - nf1[src]

    wspecs2 = [_full(D2.shape), _full(C2.shape)] + [_full(w.shape) for w in etail2]
    e2 = pl.pallas_call(
        _edge_stage2_body,
        grid=grid,
        in_specs=[pl.BlockSpec((EB, HID), lambda i: (i, 0))] * 2 + wspecs2,
        out_specs=pl.BlockSpec((EB, HID), lambda i: (i, 0)),
        out_shape=jax.ShapeDtypeStruct((E, HID), jnp.float32),
    )(d, ef1, D2, C2, *etail2)

    aggr2 = jax.ops.segment_sum(e2, dst, num_segments=N)

    vals = inv_edge_distance3 * approach_speed * edge_direction3 * scale * 10.0
    ob_raw = jnp.zeros((N, 2), jnp.float32).at[edge_index3[0]].add(vals)

    wspecsn2 = ([_full(P2.shape), _full(Q2.shape)] +
                [_full(w.shape) for w in ntail2] +
                [_full(pout['Ws'][0].shape), _full((1, HID)),
                 _full(pout['Ws'][1].shape)])
    out = pl.pallas_call(
        _node_stage2_body,
        grid=ngrid,
        in_specs=[pl.BlockSpec((NB, HID), lambda i: (i, 0))] * 2 +
                 [pl.BlockSpec((NB, 2), lambda i: (i, 0))] * 4 +
                 [pl.BlockSpec((1, 8), lambda i: (0, 0))] + wspecsn2,
        out_specs=pl.BlockSpec((NB, 2), lambda i: (i, 0)),
        out_shape=jax.ShapeDtypeStruct((N, 2), jnp.float32),
    )(nf1, aggr2, ob_raw, recent_velocity, norm_inv_distance_to_boundary,
      norm_distance_to_boundary, svec,
      P2, Q2, *ntail2, pout['Ws'][0], r(pout['bs'][0]), pout['Ws'][1])

    return out


# speedup vs baseline: 3.1745x; 2.1896x over previous
import functools
import jax
import jax.numpy as jnp
from jax import lax
from jax.experimental import pallas as pl
from jax.experimental.pallas import tpu as pltpu
from jax.experimental.pallas import tpu_sc as plsc

_EPS = 1e-5

_N = 10000
_E = 160000
_E3 = 40000
_E3P = 40960
_HID = 32
_NC, _NS = 2, 16
_NW = _NC * _NS

_TILE_E = _E // _NW
_CHUNK = 128
_NFULL = _TILE_E // _CHUNK
_REM = _TILE_E - _NFULL * _CHUNK
_NP = 10240
_NROWS = _NP // _NS
_TILE_E3 = _E3P // _NW
_NFULL3 = _TILE_E3 // _CHUNK
_OBW = 8

_MESH = functools.partial(plsc.VectorSubcoreMesh, core_axis_name="c",
                          subcore_axis_name="s", num_cores=_NC,
                          num_subcores=_NS)
_SC_PARAMS = pltpu.CompilerParams(use_tc_tiling_on_sc=False)


def _sc_segsum_body(e_hbm, dst_hbm, zeros_hbm, out_hbm, idx_v, rows_v,
                    idx_r, rows_r, acc_sh):
    c = lax.axis_index("c")
    s = lax.axis_index("s")
    row0 = s * _NROWS
    pltpu.sync_copy(zeros_hbm.at[pl.ds(row0, _NROWS)],
                    acc_sh.at[pl.ds(row0, _NROWS)])
    plsc.subcore_barrier()
    base = (c * _NS + s) * _TILE_E

    def chunk(i, _):
        off = base + i * _CHUNK
        pltpu.sync_copy(dst_hbm.at[pl.ds(off, _CHUNK)], idx_v)
        pltpu.sync_copy(e_hbm.at[pl.ds(off, _CHUNK)], rows_v)
        pltpu.sync_copy(rows_v, acc_sh.at[idx_v], add=True)
        return ()

    lax.fori_loop(0, _NFULL, chunk, (), unroll=False)
    offr = base + _NFULL * _CHUNK
    pltpu.sync_copy(dst_hbm.at[pl.ds(offr, _REM)], idx_r)
    pltpu.sync_copy(e_hbm.at[pl.ds(offr, _REM)], rows_r)
    pltpu.sync_copy(rows_r, acc_sh.at[idx_r], add=True)
    plsc.subcore_barrier()
    pltpu.sync_copy(acc_sh.at[pl.ds(row0, _NROWS)],
                    out_hbm.at[c, pl.ds(row0, _NROWS)])


_sc_segsum = pl.kernel(
    _sc_segsum_body,
    compiler_params=_SC_PARAMS,
    out_type=jax.ShapeDtypeStruct((_NC, _NP, _HID), jnp.float32),
    mesh=_MESH(),
    scratch_types=[
        pltpu.VMEM((_CHUNK,), jnp.int32),
        pltpu.VMEM((_CHUNK, _HID), jnp.float32),
        pltpu.VMEM((_REM,), jnp.int32),
        pltpu.VMEM((_REM, _HID), jnp.float32),
        pltpu.VMEM_SHARED((_NP, _HID), jnp.float32),
    ],
)


def _sc_gatherdiff_body(tab_hbm, src_hbm, dst_hbm, d_hbm, idxs_v, idxd_v,
                        rs_v, rd_v, idxs_r, idxd_r, rs_r, rd_r, tab_sh):
    c = lax.axis_index("c")
    s = lax.axis_index("s")
    row0 = s * _NROWS
    pltpu.sync_copy(tab_hbm.at[pl.ds(row0, _NROWS)],
                    tab_sh.at[pl.ds(row0, _NROWS)])
    plsc.subcore_barrier()
    base = (c * _NS + s) * _TILE_E

    def diff(dref, sref, n):
        for r in range(n):
            for k in range(_HID // 16):
                sl = pl.ds(k * 16, 16)
                dref[r, sl] = dref[r, sl] - sref[r, sl]

    def chunk(i, _):
        off = base + i * _CHUNK
        pltpu.sync_copy(src_hbm.at[pl.ds(off, _CHUNK)], idxs_v)
        pltpu.sync_copy(dst_hbm.at[pl.ds(off, _CHUNK)], idxd_v)
        pltpu.sync_copy(tab_sh.at[idxs_v], rs_v)
        pltpu.sync_copy(tab_sh.at[idxd_v], rd_v)
        diff(rd_v, rs_v, _CHUNK)
        pltpu.sync_copy(rd_v, d_hbm.at[pl.ds(off, _CHUNK)])
        return ()

    lax.fori_loop(0, _NFULL, chunk, (), unroll=False)
    offr = base + _NFULL * _CHUNK
    pltpu.sync_copy(src_hbm.at[pl.ds(offr, _REM)], idxs_r)
    pltpu.sync_copy(dst_hbm.at[pl.ds(offr, _REM)], idxd_r)
    pltpu.sync_copy(tab_sh.at[idxs_r], rs_r)
    pltpu.sync_copy(tab_sh.at[idxd_r], rd_r)
    diff(rd_r, rs_r, _REM)
    pltpu.sync_copy(rd_r, d_hbm.at[pl.ds(offr, _REM)])


_sc_gatherdiff = pl.kernel(
    _sc_gatherdiff_body,
    compiler_params=_SC_PARAMS,
    out_type=jax.ShapeDtypeStruct((_E, _HID), jnp.float32),
    mesh=_MESH(),
    scratch_types=[
        pltpu.VMEM((_CHUNK,), jnp.int32),
        pltpu.VMEM((_CHUNK,), jnp.int32),
        pltpu.VMEM((_CHUNK, _HID), jnp.float32),
        pltpu.VMEM((_CHUNK, _HID), jnp.float32),
        pltpu.VMEM((_REM,), jnp.int32),
        pltpu.VMEM((_REM,), jnp.int32),
        pltpu.VMEM((_REM, _HID), jnp.float32),
        pltpu.VMEM((_REM, _HID), jnp.float32),
        pltpu.VMEM_SHARED((_NP, _HID), jnp.float32),
    ],
)


def _sc_obscatter_body(vals_hbm, idx_hbm, zeros_hbm, out_hbm, idx_v, rows_v,
                       acc_sh):
    c = lax.axis_index("c")
    s = lax.axis_index("s")
    row0 = s * _NROWS
    pltpu.sync_copy(zeros_hbm.at[pl.ds(row0, _NROWS)],
                    acc_sh.at[pl.ds(row0, _NROWS)])
    plsc.subcore_barrier()
    base = (c * _NS + s) * _TILE_E3

    def chunk(i, _):
        off = base + i * _CHUNK
        pltpu.sync_copy(idx_hbm.at[pl.ds(off, _CHUNK)], idx_v)
        pltpu.sync_copy(vals_hbm.at[pl.ds(off, _CHUNK)], rows_v)
        pltpu.sync_copy(rows_v, acc_sh.at[idx_v], add=True)
        return ()

    lax.fori_loop(0, _NFULL3, chunk, (), unroll=False)
    plsc.subcore_barrier()
    pltpu.sync_copy(acc_sh.at[pl.ds(row0, _NROWS)],
                    out_hbm.at[c, pl.ds(row0, _NROWS)])


_sc_obscatter = pl.kernel(
    _sc_obscatter_body,
    compiler_params=_SC_PARAMS,
    out_type=jax.ShapeDtypeStruct((_NC, _NP, _OBW), jnp.float32),
    mesh=_MESH(),
    scratch_types=[
        pltpu.VMEM((_CHUNK,), jnp.int32),
        pltpu.VMEM((_CHUNK, _OBW), jnp.float32),
        pltpu.VMEM_SHARED((_NP, _OBW), jnp.float32),
    ],
)


def _dot(a, b):
    return jnp.dot(a, b, preferred_element_type=jnp.float32)


def _antisym_tail(t, b1, W2, b2, W3, b3, g):
    m = t.shape[0]
    tt = jnp.concatenate([t, -t], axis=0)
    a = jnp.maximum(tt + b1, 0.0)
    a = jnp.maximum(_dot(a, W2) + b2, 0.0)
    a = _dot(a, W3) + b3
    if g is not None:
        mu = jnp.mean(a, axis=-1, keepdims=True)
        var = jnp.mean((a - mu) ** 2, axis=-1, keepdims=True)
        a = (a - mu) * jax.lax.rsqrt(var + _EPS) * g
    return 0.5 * (a[:m] - a[m:])


def _edge_stage1_body(ea_ref,
                      eW1, eb1, eW2, eb2, eW3, eb3, eg,
                      C1, fb1, fW2, fb2, fW3, fb3, fg,
                      e1_ref, ef1_ref):
    x = ea_ref[...]
    t = _dot(x, eW1[...])
    ef = _antisym_tail(t, eb1[...], eW2[...], eb2[...], eW3[...], eb3[...], eg[...])
    t2 = _dot(ef, C1[...])
    e1 = _antisym_tail(t2, fb1[...], fW2[...], fb2[...], fW3[...], fb3[...], fg[...])
    e1_ref[...] = e1
    ef1_ref[...] = ef + e1


def _edge_stage2_body(td_ref, ef1_ref,
                      C2, fb1, fW2, fb2, fW3, fb3, fg,
                      e2_ref):
    t = td_ref[...] + _dot(ef1_ref[...], C2[...])
    e2_ref[...] = _antisym_tail(t, fb1[...], fW2[...], fb2[...], fW3[...], fb3[...], fg[...])


def _node_stage1_body(p0_ref, p1_ref, Q1, nb1, nW2, nb2, nW3, nb3, ng, D2,
                      nf1_ref, nf1d_ref):
    t = _dot(p0_ref[0] + p1_ref[0], Q1[...])
    nf1 = _antisym_tail(t, nb1[...], nW2[...], nb2[...], nW3[...], nb3[...], ng[...])
    nf1_ref[...] = nf1
    nf1d_ref[...] = _dot(nf1, D2[...])


def _vals3_body(inv_ref, app_ref, dir_ref, svec_ref, out_ref):
    s10 = svec_ref[:, 7:8]
    v2 = inv_ref[...] * app_ref[...] * dir_ref[...] * s10
    out_ref[...] = jnp.concatenate(
        [v2, jnp.zeros((v2.shape[0], _OBW - 2), v2.dtype)], axis=1)


def _node_stage2_body(nf1_ref, a0_ref, a1_ref, ob0_ref, ob1_ref, rv_ref,
                      nidb_ref, ndb_ref, svec_ref,
                      P2, Q2, nb1, nW2, nb2, nW3, nb3, ng,
                      oW1, ob1w, oW2,
                      out_ref):
    nf1 = nf1_ref[...]
    t = _dot(nf1, P2[...]) + _dot(a0_ref[0] + a1_ref[0], Q2[...])
    nf2 = nf1 + _antisym_tail(t, nb1[...], nW2[...], nb2[...], nW3[...], nb3[...], ng[...])
    t3 = _dot(nf2, oW1[...])
    swarm = 0.5 * _dot(jnp.maximum(t3 + ob1w[...], 0.0) - jnp.maximum(-t3 + ob1w[...], 0.0),
                       oW2[...])
    sv = svec_ref[...]
    gdown = sv[:, 0:2]
    scale = sv[:, 2:3]
    ux = sv[:, 3:5]
    uy = sv[:, 5:7]
    out = gdown + swarm
    rv = rv_ref[...]
    ob_raw = ob0_ref[0][:, 0:2] + ob1_ref[0][:, 0:2]
    cap = jnp.abs(rv * scale)
    out = out - jnp.sign(ob_raw) * jnp.minimum(jnp.abs(ob_raw), cap)
    nidb = nidb_ref[...]
    ndb = ndb_ref[...]
    wall = jnp.zeros_like(out)
    for axis, u in ((0, ux), (1, uy)):
        col = slice(axis, axis + 1)
        active = nidb[:, col] > 1e-07
        neg = ndb[:, col] < 0
        v = jnp.clip(-rv[:, col], 0.0, 1000.0)
        contrib_neg = v * u
        contrib_pos = nidb[:, col] * v * u
        wall = wall + jnp.where(active & neg, contrib_neg, 0.0)
        wall = wall + jnp.where(active & (~neg), contrib_pos, 0.0)
    out_ref[...] = out - wall


def _full(shape):
    return pl.BlockSpec(shape, lambda *_: tuple(0 for _ in shape))


def kernel(x, edge_index, edge_attr, edge_index3, recent_velocity, unit_x,
           unit_y, down_direction, approach_speed, inv_edge_distance3,
           edge_direction3, norm_inv_distance_to_boundary,
           norm_distance_to_boundary, acceleration_scale, velocity_scale,
           params):
    N, E, E3, HID = _N, _E, _E3, _HID

    pe_in = params['edge_in1']
    l1, l2 = params['layers1']
    pout = params['node_out1']

    def r(b):
        return b.reshape(1, -1)

    ein = (pe_in['Ws'][0], r(pe_in['bs'][0]), pe_in['Ws'][1], r(pe_in['bs'][1]),
           pe_in['Ws'][2], r(pe_in['bs'][2]), r(pe_in['g']))

    def edge_layer_w(p):
        W = p['edge']['Ws'][0]
        D = W[:HID] - W[HID:2 * HID]
        C = W[2 * HID:]
        tail = (r(p['edge']['bs'][0]), p['edge']['Ws'][1], r(p['edge']['bs'][1]),
                p['edge']['Ws'][2], r(p['edge']['bs'][2]), r(p['edge']['g']))
        return D, C, tail

    def node_layer_w(p):
        W = p['node']['Ws'][0]
        P = W[:HID]
        Q = W[HID:]
        tail = (r(p['node']['bs'][0]), p['node']['Ws'][1], r(p['node']['bs'][1]),
                p['node']['Ws'][2], r(p['node']['bs'][2]), r(p['node']['g']))
        return P, Q, tail

    _, C1, etail1 = edge_layer_w(l1)
    D2, C2, etail2 = edge_layer_w(l2)
    _, Q1, ntail1 = node_layer_w(l1)
    P2, Q2, ntail2 = node_layer_w(l2)

    g_const = 5.5339e-05 / acceleration_scale
    scale = velocity_scale / acceleration_scale * 60.0 / 94.0
    svec = jnp.concatenate([
        (g_const * down_direction).reshape(2),
        scale.reshape(1),
        (-unit_x * scale).reshape(2),
        (-unit_y * scale).reshape(2),
        (scale * 10.0).reshape(1),
    ]).reshape(1, 8)

    src, dst = edge_index[0], edge_index[1]
    zeros32 = jnp.zeros((_NP, HID), jnp.float32)
    zeros8 = jnp.zeros((_NP, _OBW), jnp.float32)

    EB = 4000
    grid = (E // EB,)
    wspecs1 = [_full(w.shape) for w in ein] + [_full(C1.shape)] + \
              [_full(w.shape) for w in etail1]
    e1, ef1 = pl.pallas_call(
        _edge_stage1_body,
        grid=grid,
        in_specs=[pl.BlockSpec((EB, 14), lambda i: (i, 0))] + wspecs1,
        out_specs=[pl.BlockSpec((EB, HID), lambda i: (i, 0))] * 2,
        out_shape=[jax.ShapeDtypeStruct((E, HID), jnp.float32)] * 2,
    )(edge_attr, *ein, C1, *etail1)

    aggr1 = _sc_segsum(e1, dst, zeros32)

    NB = 2000
    ngrid = (N // NB,)
    wspecsn1 = [_full(Q1.shape)] + [_full(w.shape) for w in ntail1] + \
               [_full(D2.shape)]
    nf1, nf1d = pl.pallas_call(
        _node_stage1_body,
        grid=ngrid,
        in_specs=[pl.BlockSpec((1, NB, HID), lambda i: (0, i, 0)),
                  pl.BlockSpec((1, NB, HID), lambda i: (1, i, 0))] + wspecsn1,
        out_specs=[pl.BlockSpec((NB, HID), lambda i: (i, 0))] * 2,
        out_shape=[jax.ShapeDtypeStruct((N, HID), jnp.float32),
                   jax.ShapeDtypeStruct((_NP, HID), jnp.float32)],
    )(aggr1, aggr1, Q1, *ntail1, D2)

    td = _sc_gatherdiff(nf1d, src, dst)

    wspecs2 = [_full(C2.shape)] + [_full(w.shape) for w in etail2]
    e2 = pl.pallas_call(
        _edge_stage2_body,
        grid=grid,
        in_specs=[pl.BlockSpec((EB, HID), lambda i: (i, 0))] * 2 + wspecs2,
        out_specs=pl.BlockSpec((EB, HID), lambda i: (i, 0)),
        out_shape=jax.ShapeDtypeStruct((E, HID), jnp.float32),
    )(td, ef1, C2, *etail2)

    aggr2 = _sc_segsum(e2, dst, zeros32)

    VB = 8192
    vgrid = (_E3P // VB,)
    pad3 = _E3P - E3
    inv_p = jnp.pad(inv_edge_distance3, ((0, pad3), (0, 0)))
    app_p = jnp.pad(approach_speed, ((0, pad3), (0, 0)))
    dir_p = jnp.pad(edge_direction3, ((0, pad3), (0, 0)))
    idx3_p = jnp.pad(edge_index3[0], ((0, pad3),))
    vals8 = pl.pallas_call(
        _vals3_body,
        grid=vgrid,
        in_specs=[pl.BlockSpec((VB, 1), lambda i: (i, 0)),
                  pl.BlockSpec((VB, 1), lambda i: (i, 0)),
                  pl.BlockSpec((VB, 2), lambda i: (i, 0)),
                  _full((1, 8))],
        out_specs=pl.BlockSpec((VB, _OBW), lambda i: (i, 0)),
        out_shape=jax.ShapeDtypeStruct((_E3P, _OBW), jnp.float32),
    )(inv_p, app_p, dir_p, svec)
    ob_part = _sc_obscatter(vals8, idx3_p, zeros8)

    wspecsn2 = ([_full(P2.shape), _full(Q2.shape)] +
                [_full(w.shape) for w in ntail2] +
                [_full(pout['Ws'][0].shape), _full((1, HID)),
                 _full(pout['Ws'][1].shape)])
    a3 = aggr2
    ob3 = ob_part
    out = pl.pallas_call(
        _node_stage2_body,
        grid=ngrid,
        in_specs=[pl.BlockSpec((NB, HID), lambda i: (i, 0)),
                  pl.BlockSpec((1, NB, HID), lambda i: (0, i, 0)),
                  pl.BlockSpec((1, NB, HID), lambda i: (1, i, 0)),
                  pl.BlockSpec((1, NB, _OBW), lambda i: (0, i, 0)),
                  pl.BlockSpec((1, NB, _OBW), lambda i: (1, i, 0))] +
                 [pl.BlockSpec((NB, 2), lambda i: (i, 0))] * 3 +
                 [_full((1, 8))] + wspecsn2,
        out_specs=pl.BlockSpec((NB, 2), lambda i: (i, 0)),
        out_shape=jax.ShapeDtypeStruct((N, 2), jnp.float32),
    )(nf1, a3, a3, ob3, ob3, recent_velocity, norm_inv_distance_to_boundary,
      norm_distance_to_boundary, svec,
      P2, Q2, *ntail2, pout['Ws'][0], r(pout['bs'][0]), pout['Ws'][1])

    return out


# speedup vs baseline: 4.3584x; 1.3729x over previous
import functools
import jax
import jax.numpy as jnp
from jax import lax
from jax.experimental import pallas as pl
from jax.experimental.pallas import tpu as pltpu
from jax.experimental.pallas import tpu_sc as plsc
from jax.scipy.linalg import block_diag

_EPS = 1e-5

_N = 10000
_E = 160000
_E3 = 40000
_E3P = 40960
_HID = 32
_PK = 8
_NC, _NS = 2, 16
_NW = _NC * _NS

_TILE_E = _E // _NW
_CHUNK = 128
_NFULL = _TILE_E // _CHUNK
_REM = _TILE_E - _NFULL * _CHUNK
_NP = 10240
_NP8 = _NP // _PK
_NROWS = _NP // _NS
_TILE_E3 = _E3P // _NW
_NFULL3 = _TILE_E3 // _CHUNK
_OBW = 8

_MESH = functools.partial(plsc.VectorSubcoreMesh, core_axis_name="c",
                          subcore_axis_name="s", num_cores=_NC,
                          num_subcores=_NS)
_SC_PARAMS = pltpu.CompilerParams(use_tc_tiling_on_sc=False)


def _sc_segsum_body(e_hbm, dst_hbm, zeros_hbm, out_hbm, idx_v, rows_v,
                    idx_r, rows_r, acc_sh):
    c = lax.axis_index("c")
    s = lax.axis_index("s")
    row0 = s * _NROWS
    pltpu.sync_copy(zeros_hbm.at[pl.ds(row0, _NROWS)],
                    acc_sh.at[pl.ds(row0, _NROWS)])
    plsc.subcore_barrier()
    base = (c * _NS + s) * _TILE_E

    def chunk(i, _):
        off = base + i * _CHUNK
        pltpu.sync_copy(dst_hbm.at[pl.ds(off, _CHUNK)], idx_v)
        pltpu.sync_copy(e_hbm.at[pl.ds(off, _CHUNK)], rows_v)
        pltpu.sync_copy(rows_v, acc_sh.at[idx_v], add=True)
        return ()

    lax.fori_loop(0, _NFULL, chunk, (), unroll=False)
    offr = base + _NFULL * _CHUNK
    pltpu.sync_copy(dst_hbm.at[pl.ds(offr, _REM)], idx_r)
    pltpu.sync_copy(e_hbm.at[pl.ds(offr, _REM)], rows_r)
    pltpu.sync_copy(rows_r, acc_sh.at[idx_r], add=True)
    plsc.subcore_barrier()
    pltpu.sync_copy(acc_sh.at[pl.ds(row0, _NROWS)],
                    out_hbm.at[c, pl.ds(row0, _NROWS)])


def _sc_gatherdiff_body(tab_hbm, src_hbm, dst_hbm, d_hbm, idxs_v, idxd_v,
                        rs_v, rd_v, idxs_r, idxd_r, rs_r, rd_r, tab_sh):
    c = lax.axis_index("c")
    s = lax.axis_index("s")
    row0 = s * _NROWS
    pltpu.sync_copy(tab_hbm.at[pl.ds(row0, _NROWS)],
                    tab_sh.at[pl.ds(row0, _NROWS)])
    plsc.subcore_barrier()
    base = (c * _NS + s) * _TILE_E

    def diff(dref, sref, n):
        for r in range(n):
            for k in range(_HID // 16):
                sl = pl.ds(k * 16, 16)
                dref[r, sl] = dref[r, sl] - sref[r, sl]

    def chunk(i, _):
        off = base + i * _CHUNK
        pltpu.sync_copy(src_hbm.at[pl.ds(off, _CHUNK)], idxs_v)
        pltpu.sync_copy(dst_hbm.at[pl.ds(off, _CHUNK)], idxd_v)
        pltpu.sync_copy(tab_sh.at[idxs_v], rs_v)
        pltpu.sync_copy(tab_sh.at[idxd_v], rd_v)
        diff(rd_v, rs_v, _CHUNK)
        pltpu.sync_copy(rd_v, d_hbm.at[pl.ds(off, _CHUNK)])
        return ()

    lax.fori_loop(0, _NFULL, chunk, (), unroll=False)
    offr = base + _NFULL * _CHUNK
    pltpu.sync_copy(src_hbm.at[pl.ds(offr, _REM)], idxs_r)
    pltpu.sync_copy(dst_hbm.at[pl.ds(offr, _REM)], idxd_r)
    pltpu.sync_copy(tab_sh.at[idxs_r], rs_r)
    pltpu.sync_copy(tab_sh.at[idxd_r], rd_r)
    diff(rd_r, rs_r, _REM)
    pltpu.sync_copy(rd_r, d_hbm.at[pl.ds(offr, _REM)])


def _sc_obscatter_body(vals_hbm, idx_hbm, zeros_hbm, out_hbm, idx_v, rows_v,
                       acc_sh):
    c = lax.axis_index("c")
    s = lax.axis_index("s")
    row0 = s * _NROWS
    pltpu.sync_copy(zeros_hbm.at[pl.ds(row0, _NROWS)],
                    acc_sh.at[pl.ds(row0, _NROWS)])
    plsc.subcore_barrier()
    base = (c * _NS + s) * _TILE_E3

    def chunk(i, _):
        off = base + i * _CHUNK
        pltpu.sync_copy(idx_hbm.at[pl.ds(off, _CHUNK)], idx_v)
        pltpu.sync_copy(vals_hbm.at[pl.ds(off, _CHUNK)], rows_v)
        pltpu.sync_copy(rows_v, acc_sh.at[idx_v], add=True)
        return ()

    lax.fori_loop(0, _NFULL3, chunk, (), unroll=False)
    plsc.subcore_barrier()
    pltpu.sync_copy(acc_sh.at[pl.ds(row0, _NROWS)],
                    out_hbm.at[c, pl.ds(row0, _NROWS)])


@functools.lru_cache(maxsize=None)
def _sc_kernels():
    mesh = _MESH()
    seg = pl.kernel(
        _sc_segsum_body,
        compiler_params=_SC_PARAMS,
        out_type=jax.ShapeDtypeStruct((_NC, _NP, _HID), jnp.float32),
        mesh=mesh,
        scratch_types=[
            pltpu.VMEM((_CHUNK,), jnp.int32),
            pltpu.VMEM((_CHUNK, _HID), jnp.float32),
            pltpu.VMEM((_REM,), jnp.int32),
            pltpu.VMEM((_REM, _HID), jnp.float32),
            pltpu.VMEM_SHARED((_NP, _HID), jnp.float32),
        ],
    )
    gd = pl.kernel(
        _sc_gatherdiff_body,
        compiler_params=_SC_PARAMS,
        out_type=jax.ShapeDtypeStruct((_E, _HID), jnp.float32),
        mesh=mesh,
        scratch_types=[
            pltpu.VMEM((_CHUNK,), jnp.int32),
            pltpu.VMEM((_CHUNK,), jnp.int32),
            pltpu.VMEM((_CHUNK, _HID), jnp.float32),
            pltpu.VMEM((_CHUNK, _HID), jnp.float32),
            pltpu.VMEM((_REM,), jnp.int32),
            pltpu.VMEM((_REM,), jnp.int32),
            pltpu.VMEM((_REM, _HID), jnp.float32),
            pltpu.VMEM((_REM, _HID), jnp.float32),
            pltpu.VMEM_SHARED((_NP, _HID), jnp.float32),
        ],
    )
    ob = pl.kernel(
        _sc_obscatter_body,
        compiler_params=_SC_PARAMS,
        out_type=jax.ShapeDtypeStruct((_NC, _NP, _OBW), jnp.float32),
        mesh=mesh,
        scratch_types=[
            pltpu.VMEM((_CHUNK,), jnp.int32),
            pltpu.VMEM((_CHUNK, _OBW), jnp.float32),
            pltpu.VMEM_SHARED((_NP, _OBW), jnp.float32),
        ],
    )
    return seg, gd, ob


def _sc_segsum(e, dst, zeros):
    return _sc_kernels()[0](e, dst, zeros)


def _sc_gatherdiff(tab, src, dst):
    return _sc_kernels()[1](tab, src, dst)


def _sc_obscatter(vals, idx, zeros):
    return _sc_kernels()[2](vals, idx, zeros)


def _dot(a, b):
    return jnp.dot(a, b, preferred_element_type=jnp.float32)


def _antisym_tail_p(t, b1, W2, b2, W3, b3, g, M):
    m = t.shape[0]
    tt = jnp.concatenate([t, -t], axis=0)
    a = jnp.maximum(tt + b1, 0.0)
    a = jnp.maximum(_dot(a, W2) + b2, 0.0)
    a = _dot(a, W3) + b3
    if g is not None:
        mu = _dot(a, M)
        var = _dot(a * a, M) - mu * mu
        a = (a - mu) * jax.lax.rsqrt(jnp.maximum(var, 0.0) + _EPS) * g
    return 0.5 * (a[:m] - a[m:])


def _edge_stage1_body(ea_ref, M,
                      eW1, eb1, eW2, eb2, eW3, eb3, eg,
                      C1, fb1, fW2, fb2, fW3, fb3, fg,
                      e1_ref, ef1_ref):
    t = _dot(ea_ref[...], eW1[...])
    ef = _antisym_tail_p(t, eb1[...], eW2[...], eb2[...], eW3[...], eb3[...],
                         eg[...], M[...])
    t2 = _dot(ef, C1[...])
    e1 = _antisym_tail_p(t2, fb1[...], fW2[...], fb2[...], fW3[...], fb3[...],
                         fg[...], M[...])
    e1_ref[...] = e1
    ef1_ref[...] = ef + e1


def _edge_stage2_body(td_ref, ef1_ref, M,
                      C2, fb1, fW2, fb2, fW3, fb3, fg,
                      e2_ref):
    t = td_ref[...] + _dot(ef1_ref[...], C2[...])
    e2_ref[...] = _antisym_tail_p(t, fb1[...], fW2[...], fb2[...], fW3[...],
                                  fb3[...], fg[...], M[...])


def _node_stage1_body(p0_ref, p1_ref, M, Q1, nb1, nW2, nb2, nW3, nb3, ng, D2,
                      nf1_ref, nf1d_ref):
    t = _dot(p0_ref[0] + p1_ref[0], Q1[...])
    nf1 = _antisym_tail_p(t, nb1[...], nW2[...], nb2[...], nW3[...], nb3[...],
                          ng[...], M[...])
    nf1_ref[...] = nf1
    nf1d_ref[...] = _dot(nf1, D2[...])


def _node_stage2_body(nf1_ref, a0_ref, a1_ref, M,
                      P2, Q2, nb1, nW2, nb2, nW3, nb3, ng,
                      oW1, ob1w, oW2,
                      swarm_ref):
    nf1 = nf1_ref[...]
    t = _dot(nf1, P2[...]) + _dot(a0_ref[0] + a1_ref[0], Q2[...])
    nf2 = nf1 + _antisym_tail_p(t, nb1[...], nW2[...], nb2[...], nW3[...],
                                nb3[...], ng[...], M[...])
    t3 = _dot(nf2, oW1[...])
    swarm_ref[...] = 0.5 * _dot(
        jnp.maximum(t3 + ob1w[...], 0.0) - jnp.maximum(-t3 + ob1w[...], 0.0),
        oW2[...])


def _vals3_body(inv_ref, app_ref, dir_ref, svec_ref, out_ref):
    s10 = svec_ref[:, 7:8]
    v2 = inv_ref[...] * app_ref[...] * dir_ref[...] * s10
    out_ref[...] = jnp.concatenate(
        [v2, jnp.zeros((v2.shape[0], _OBW - 2), v2.dtype)], axis=1)


def _assemble_body(swarm_ref, ob0_ref, ob1_ref, rv_ref, nidb_ref, ndb_ref,
                   svec_ref, out_ref):
    sv = svec_ref[...]
    gdown = sv[:, 0:2]
    scale = sv[:, 2:3]
    ux = sv[:, 3:5]
    uy = sv[:, 5:7]
    out = gdown + swarm_ref[...]
    rv = rv_ref[...]
    ob_raw = ob0_ref[0][:, 0:2] + ob1_ref[0][:, 0:2]
    cap = jnp.abs(rv * scale)
    out = out - jnp.sign(ob_raw) * jnp.minimum(jnp.abs(ob_raw), cap)
    nidb = nidb_ref[...]
    ndb = ndb_ref[...]
    wall = jnp.zeros_like(out)
    for axis, u in ((0, ux), (1, uy)):
        col = slice(axis, axis + 1)
        active = nidb[:, col] > 1e-07
        neg = ndb[:, col] < 0
        v = jnp.clip(-rv[:, col], 0.0, 1000.0)
        contrib_neg = v * u
        contrib_pos = nidb[:, col] * v * u
        wall = wall + jnp.where(active & neg, contrib_neg, 0.0)
        wall = wall + jnp.where(active & (~neg), contrib_pos, 0.0)
    out_ref[...] = out - wall


def _full(shape):
    return pl.BlockSpec(shape, lambda *_: tuple(0 for _ in shape))


def _bd8(W):
    return block_diag(*([W] * _PK))


def kernel(x, edge_index, edge_attr, edge_index3, recent_velocity, unit_x,
           unit_y, down_direction, approach_speed, inv_edge_distance3,
           edge_direction3, norm_inv_distance_to_boundary,
           norm_distance_to_boundary, acceleration_scale, velocity_scale,
           params):
    N, E, E3, HID = _N, _E, _E3, _HID
    E8 = E // _PK
    W256 = HID * _PK

    pe_in = params['edge_in1']
    l1, l2 = params['layers1']
    pout = params['node_out1']

    def rt(b):
        return jnp.tile(b.reshape(1, -1), (1, _PK))

    Mavg = _bd8(jnp.full((HID, HID), 1.0 / HID, jnp.float32))

    ein = (_bd8(pe_in['Ws'][0]), rt(pe_in['bs'][0]), _bd8(pe_in['Ws'][1]),
           rt(pe_in['bs'][1]), _bd8(pe_in['Ws'][2]), rt(pe_in['bs'][2]),
           rt(pe_in['g']))

    def edge_layer_w(p):
        W = p['edge']['Ws'][0]
        D = _bd8(W[:HID] - W[HID:2 * HID])
        C = _bd8(W[2 * HID:])
        tail = (rt(p['edge']['bs'][0]), _bd8(p['edge']['Ws'][1]),
                rt(p['edge']['bs'][1]), _bd8(p['edge']['Ws'][2]),
                rt(p['edge']['bs'][2]), rt(p['edge']['g']))
        return D, C, tail

    def node_layer_w(p):
        W = p['node']['Ws'][0]
        P = _bd8(W[:HID])
        Q = _bd8(W[HID:])
        tail = (rt(p['node']['bs'][0]), _bd8(p['node']['Ws'][1]),
                rt(p['node']['bs'][1]), _bd8(p['node']['Ws'][2]),
                rt(p['node']['bs'][2]), rt(p['node']['g']))
        return P, Q, tail

    _, C1, etail1 = edge_layer_w(l1)
    D2, C2, etail2 = edge_layer_w(l2)
    _, Q1, ntail1 = node_layer_w(l1)
    P2, Q2, ntail2 = node_layer_w(l2)
    oW1 = _bd8(pout['Ws'][0])
    ob1w = rt(pout['bs'][0])
    oW2 = _bd8(pout['Ws'][1])

    g_const = 5.5339e-05 / acceleration_scale
    scale = velocity_scale / acceleration_scale * 60.0 / 94.0
    svec = jnp.concatenate([
        (g_const * down_direction).reshape(2),
        scale.reshape(1),
        (-unit_x * scale).reshape(2),
        (-unit_y * scale).reshape(2),
        (scale * 10.0).reshape(1),
    ]).reshape(1, 8)

    src, dst = edge_index[0], edge_index[1]
    zeros32 = jnp.zeros((_NP, HID), jnp.float32)
    zeros8 = jnp.zeros((_NP, _OBW), jnp.float32)

    EB8 = 2000
    grid = (E8 // EB8,)
    ea_p = edge_attr.reshape(E8, 14 * _PK)
    wspecs1 = [_full((W256, W256))] + [_full(w.shape) for w in ein] + \
              [_full(C1.shape)] + [_full(w.shape) for w in etail1]
    e1p, ef1p = pl.pallas_call(
        _edge_stage1_body,
        grid=grid,
        in_specs=[pl.BlockSpec((EB8, 14 * _PK), lambda i: (i, 0))] + wspecs1,
        out_specs=[pl.BlockSpec((EB8, W256), lambda i: (i, 0))] * 2,
        out_shape=[jax.ShapeDtypeStruct((E8, W256), jnp.float32)] * 2,
    )(ea_p, Mavg, *ein, C1, *etail1)

    aggr1 = _sc_segsum(e1p.reshape(E, HID), dst, zeros32)

    a1p = aggr1.reshape(_NC, _NP8, W256)
    wspecsn1 = [_full((W256, W256)), _full(Q1.shape)] + \
               [_full(w.shape) for w in ntail1] + [_full(D2.shape)]
    nf1p, nf1d = pl.pallas_call(
        _node_stage1_body,
        grid=(1,),
        in_specs=[pl.BlockSpec((1, _NP8, W256), lambda i: (0, 0, 0)),
                  pl.BlockSpec((1, _NP8, W256), lambda i: (1, 0, 0))] + wspecsn1,
        out_specs=[pl.BlockSpec((_NP8, W256), lambda i: (0, 0))] * 2,
        out_shape=[jax.ShapeDtypeStruct((_NP8, W256), jnp.float32)] * 2,
    )(a1p, a1p, Mavg, Q1, *ntail1, D2)

    td = _sc_gatherdiff(nf1d.reshape(_NP, HID), src, dst)

    wspecs2 = [_full((W256, W256)), _full(C2.shape)] + \
              [_full(w.shape) for w in etail2]
    e2p = pl.pallas_call(
        _edge_stage2_body,
        grid=grid,
        in_specs=[pl.BlockSpec((EB8, W256), lambda i: (i, 0))] * 2 + wspecs2,
        out_specs=pl.BlockSpec((EB8, W256), lambda i: (i, 0)),
        out_shape=jax.ShapeDtypeStruct((E8, W256), jnp.float32),
    )(td.reshape(E8, W256), ef1p, Mavg, C2, *etail2)

    aggr2 = _sc_segsum(e2p.reshape(E, HID), dst, zeros32)

    VB = 8192
    vgrid = (_E3P // VB,)
    pad3 = _E3P - E3
    inv_p = jnp.pad(inv_edge_distance3, ((0, pad3), (0, 0)))
    app_p = jnp.pad(approach_speed, ((0, pad3), (0, 0)))
    dir_p = jnp.pad(edge_direction3, ((0, pad3), (0, 0)))
    idx3_p = jnp.pad(edge_index3[0], ((0, pad3),))
    vals8 = pl.pallas_call(
        _vals3_body,
        grid=vgrid,
        in_specs=[pl.BlockSpec((VB, 1), lambda i: (i, 0)),
                  pl.BlockSpec((VB, 1), lambda i: (i, 0)),
                  pl.BlockSpec((VB, 2), lambda i: (i, 0)),
                  _full((1, 8))],
        out_specs=pl.BlockSpec((VB, _OBW), lambda i: (i, 0)),
        out_shape=jax.ShapeDtypeStruct((_E3P, _OBW), jnp.float32),
    )(inv_p, app_p, dir_p, svec)
    ob_part = _sc_obscatter(vals8, idx3_p, zeros8)

    a2p = aggr2.reshape(_NC, _NP8, W256)
    wspecsn2 = ([_full((W256, W256)), _full(P2.shape), _full(Q2.shape)] +
                [_full(w.shape) for w in ntail2] +
                [_full(oW1.shape), _full(ob1w.shape), _full(oW2.shape)])
    swarm = pl.pallas_call(
        _node_stage2_body,
        grid=(1,),
        in_specs=[pl.BlockSpec((_NP8, W256), lambda i: (0, 0)),
                  pl.BlockSpec((1, _NP8, W256), lambda i: (0, 0, 0)),
                  pl.BlockSpec((1, _NP8, W256), lambda i: (1, 0, 0))] + wspecsn2,
        out_specs=pl.BlockSpec((_NP8, 2 * _PK), lambda i: (0, 0)),
        out_shape=jax.ShapeDtypeStruct((_NP8, 2 * _PK), jnp.float32),
    )(nf1p, a2p, a2p, Mavg, P2, Q2, *ntail2, oW1, ob1w, oW2)

    NB = 2000
    ngrid = (N // NB,)
    out = pl.pallas_call(
        _assemble_body,
        grid=ngrid,
        in_specs=[pl.BlockSpec((NB, 2), lambda i: (i, 0)),
                  pl.BlockSpec((1, NB, _OBW), lambda i: (0, i, 0)),
                  pl.BlockSpec((1, NB, _OBW), lambda i: (1, i, 0))] +
                 [pl.BlockSpec((NB, 2), lambda i: (i, 0))] * 3 +
                 [_full((1, 8))],
        out_specs=pl.BlockSpec((NB, 2), lambda i: (i, 0)),
        out_shape=jax.ShapeDtypeStruct((N, 2), jnp.float32),
    )(swarm.reshape(_NP, 2), ob_part, ob_part, recent_velocity,
      norm_inv_distance_to_boundary, norm_distance_to_boundary, svec)

    return out


# speedup vs baseline: 4.4395x; 1.0186x over previous
import functools
import jax
import jax.numpy as jnp
from jax import lax
from jax.experimental import pallas as pl
from jax.experimental.pallas import tpu as pltpu
from jax.experimental.pallas import tpu_sc as plsc
from jax.scipy.linalg import block_diag

_EPS = 1e-5

_N = 10000
_E = 160000
_E3 = 40000
_E3P = 40960
_HID = 32
_PK = 8
_NC, _NS = 2, 16
_NW = _NC * _NS

_TILE_E = _E // _NW
_CHUNK = 128
_NFULL = _TILE_E // _CHUNK
_REM = _TILE_E - _NFULL * _CHUNK
_NP = 10240
_NP8 = _NP // _PK
_NROWS = _NP // _NS
_TILE_E3 = _E3P // _NW
_NFULL3 = _TILE_E3 // _CHUNK
_OBW = 8

_MESH = functools.partial(plsc.VectorSubcoreMesh, core_axis_name="c",
                          subcore_axis_name="s", num_cores=_NC,
                          num_subcores=_NS)
_SC_PARAMS = pltpu.CompilerParams(use_tc_tiling_on_sc=False)


def _sc_segsum_body(e_hbm, dst_hbm, zeros_hbm, out_hbm, idx_v, rows_v,
                    idx_r, rows_r, acc_sh):
    c = lax.axis_index("c")
    s = lax.axis_index("s")
    row0 = s * _NROWS
    pltpu.sync_copy(zeros_hbm.at[pl.ds(row0, _NROWS)],
                    acc_sh.at[pl.ds(row0, _NROWS)])
    plsc.subcore_barrier()
    base = (c * _NS + s) * _TILE_E

    def chunk(i, _):
        off = base + i * _CHUNK
        pltpu.sync_copy(dst_hbm.at[pl.ds(off, _CHUNK)], idx_v)
        pltpu.sync_copy(e_hbm.at[pl.ds(off, _CHUNK)], rows_v)
        pltpu.sync_copy(rows_v, acc_sh.at[idx_v], add=True)
        return ()

    lax.fori_loop(0, _NFULL, chunk, (), unroll=False)
    offr = base + _NFULL * _CHUNK
    pltpu.sync_copy(dst_hbm.at[pl.ds(offr, _REM)], idx_r)
    pltpu.sync_copy(e_hbm.at[pl.ds(offr, _REM)], rows_r)
    pltpu.sync_copy(rows_r, acc_sh.at[idx_r], add=True)
    plsc.subcore_barrier()
    pltpu.sync_copy(acc_sh.at[pl.ds(row0, _NROWS)],
                    out_hbm.at[c, pl.ds(row0, _NROWS)])


def _sc_gatherdiff_body(tab_hbm, src_hbm, dst_hbm, d_hbm, idxs_v, idxd_v,
                        rs_v, rd_v, idxs_r, idxd_r, rs_r, rd_r, tab_sh):
    c = lax.axis_index("c")
    s = lax.axis_index("s")
    row0 = s * _NROWS
    pltpu.sync_copy(tab_hbm.at[pl.ds(row0, _NROWS)],
                    tab_sh.at[pl.ds(row0, _NROWS)])
    plsc.subcore_barrier()
    base = (c * _NS + s) * _TILE_E

    def diff(dref, sref, n):
        for r in range(n):
            for k in range(_HID // 16):
                sl = pl.ds(k * 16, 16)
                dref[r, sl] = dref[r, sl] - sref[r, sl]

    def chunk(i, _):
        off = base + i * _CHUNK
        pltpu.sync_copy(src_hbm.at[pl.ds(off, _CHUNK)], idxs_v)
        pltpu.sync_copy(dst_hbm.at[pl.ds(off, _CHUNK)], idxd_v)
        pltpu.sync_copy(tab_sh.at[idxs_v], rs_v)
        pltpu.sync_copy(tab_sh.at[idxd_v], rd_v)
        diff(rd_v, rs_v, _CHUNK)
        pltpu.sync_copy(rd_v, d_hbm.at[pl.ds(off, _CHUNK)])
        return ()

    lax.fori_loop(0, _NFULL, chunk, (), unroll=False)
    offr = base + _NFULL * _CHUNK
    pltpu.sync_copy(src_hbm.at[pl.ds(offr, _REM)], idxs_r)
    pltpu.sync_copy(dst_hbm.at[pl.ds(offr, _REM)], idxd_r)
    pltpu.sync_copy(tab_sh.at[idxs_r], rs_r)
    pltpu.sync_copy(tab_sh.at[idxd_r], rd_r)
    diff(rd_r, rs_r, _REM)
    pltpu.sync_copy(rd_r, d_hbm.at[pl.ds(offr, _REM)])


def _sc_obscatter_body(vals_hbm, idx_hbm, zeros_hbm, out_hbm, idx_v, rows_v,
                       acc_sh):
    c = lax.axis_index("c")
    s = lax.axis_index("s")
    row0 = s * _NROWS
    pltpu.sync_copy(zeros_hbm.at[pl.ds(row0, _NROWS)],
                    acc_sh.at[pl.ds(row0, _NROWS)])
    plsc.subcore_barrier()
    base = (c * _NS + s) * _TILE_E3

    def chunk(i, _):
        off = base + i * _CHUNK
        pltpu.sync_copy(idx_hbm.at[pl.ds(off, _CHUNK)], idx_v)
        pltpu.sync_copy(vals_hbm.at[pl.ds(off, _CHUNK)], rows_v)
        pltpu.sync_copy(rows_v, acc_sh.at[idx_v], add=True)
        return ()

    lax.fori_loop(0, _NFULL3, chunk, (), unroll=False)
    plsc.subcore_barrier()
    pltpu.sync_copy(acc_sh.at[pl.ds(row0, _NROWS)],
                    out_hbm.at[c, pl.ds(row0, _NROWS)])


@functools.lru_cache(maxsize=None)
def _sc_kernels():
    mesh = _MESH()
    seg = pl.kernel(
        _sc_segsum_body,
        compiler_params=_SC_PARAMS,
        out_type=jax.ShapeDtypeStruct((_NC, _NP, _HID), jnp.float32),
        mesh=mesh,
        scratch_types=[
            pltpu.VMEM((_CHUNK,), jnp.int32),
            pltpu.VMEM((_CHUNK, _HID), jnp.float32),
            pltpu.VMEM((_REM,), jnp.int32),
            pltpu.VMEM((_REM, _HID), jnp.float32),
            pltpu.VMEM_SHARED((_NP, _HID), jnp.float32),
        ],
    )
    gd = pl.kernel(
        _sc_gatherdiff_body,
        compiler_params=_SC_PARAMS,
        out_type=jax.ShapeDtypeStruct((_E, _HID), jnp.float32),
        mesh=mesh,
        scratch_types=[
            pltpu.VMEM((_CHUNK,), jnp.int32),
            pltpu.VMEM((_CHUNK,), jnp.int32),
            pltpu.VMEM((_CHUNK, _HID), jnp.float32),
            pltpu.VMEM((_CHUNK, _HID), jnp.float32),
            pltpu.VMEM((_REM,), jnp.int32),
            pltpu.VMEM((_REM,), jnp.int32),
            pltpu.VMEM((_REM, _HID), jnp.float32),
            pltpu.VMEM((_REM, _HID), jnp.float32),
            pltpu.VMEM_SHARED((_NP, _HID), jnp.float32),
        ],
    )
    ob = pl.kernel(
        _sc_obscatter_body,
        compiler_params=_SC_PARAMS,
        out_type=jax.ShapeDtypeStruct((_NC, _NP, _OBW), jnp.float32),
        mesh=mesh,
        scratch_types=[
            pltpu.VMEM((_CHUNK,), jnp.int32),
            pltpu.VMEM((_CHUNK, _OBW), jnp.float32),
            pltpu.VMEM_SHARED((_NP, _OBW), jnp.float32),
        ],
    )
    return seg, gd, ob


def _sc_segsum(e, dst, zeros):
    return _sc_kernels()[0](e, dst, zeros)


def _sc_gatherdiff(tab, src, dst):
    return _sc_kernels()[1](tab, src, dst)


def _sc_obscatter(vals, idx, zeros):
    return _sc_kernels()[2](vals, idx, zeros)


def _dot(a, b):
    return jnp.dot(a, b, preferred_element_type=jnp.float32)


def _antisym_tail_p(t, b1, W2, b2, W3, b3, g, M):
    m = t.shape[0]
    tt = jnp.concatenate([t, -t], axis=0)
    a = jnp.maximum(tt + b1, 0.0)
    a = jnp.maximum(_dot(a, W2) + b2, 0.0)
    a = _dot(a, W3) + b3
    if g is not None:
        mu = _dot(a, M)
        c = a - mu
        var = _dot(c * c, M)
        a = c * jax.lax.rsqrt(var + _EPS) * g
    return 0.5 * (a[:m] - a[m:])


def _edge_stage1_body(ea_ref, M,
                      eW1, eb1, eW2, eb2, eW3, eb3, eg,
                      C1, fb1, fW2, fb2, fW3, fb3, fg,
                      e1_ref, ef1_ref):
    t = _dot(ea_ref[...], eW1[...])
    ef = _antisym_tail_p(t, eb1[...], eW2[...], eb2[...], eW3[...], eb3[...],
                         eg[...], M[...])
    t2 = _dot(ef, C1[...])
    e1 = _antisym_tail_p(t2, fb1[...], fW2[...], fb2[...], fW3[...], fb3[...],
                         fg[...], M[...])
    e1_ref[...] = e1
    ef1_ref[...] = ef + e1


def _edge_stage2_body(td_ref, ef1_ref, M,
                      C2, fb1, fW2, fb2, fW3, fb3, fg,
                      e2_ref):
    t = td_ref[...] + _dot(ef1_ref[...], C2[...])
    e2_ref[...] = _antisym_tail_p(t, fb1[...], fW2[...], fb2[...], fW3[...],
                                  fb3[...], fg[...], M[...])


def _node_stage1_body(p0_ref, p1_ref, M, Q1, nb1, nW2, nb2, nW3, nb3, ng, D2,
                      nf1_ref, nf1d_ref):
    t = _dot(p0_ref[0] + p1_ref[0], Q1[...])
    nf1 = _antisym_tail_p(t, nb1[...], nW2[...], nb2[...], nW3[...], nb3[...],
                          ng[...], M[...])
    nf1_ref[...] = nf1
    nf1d_ref[...] = _dot(nf1, D2[...])


def _node_stage2_body(nf1_ref, a0_ref, a1_ref, M,
                      P2, Q2, nb1, nW2, nb2, nW3, nb3, ng,
                      oW1, ob1w, oW2,
                      swarm_ref):
    nf1 = nf1_ref[...]
    t = _dot(nf1, P2[...]) + _dot(a0_ref[0] + a1_ref[0], Q2[...])
    nf2 = nf1 + _antisym_tail_p(t, nb1[...], nW2[...], nb2[...], nW3[...],
                                nb3[...], ng[...], M[...])
    t3 = _dot(nf2, oW1[...])
    swarm_ref[...] = 0.5 * _dot(
        jnp.maximum(t3 + ob1w[...], 0.0) - jnp.maximum(-t3 + ob1w[...], 0.0),
        oW2[...])


def _vals3_body(inv_ref, app_ref, dir_ref, svec_ref, out_ref):
    s10 = svec_ref[:, 7:8]
    v2 = inv_ref[...] * app_ref[...] * dir_ref[...] * s10
    out_ref[...] = jnp.concatenate(
        [v2, jnp.zeros((v2.shape[0], _OBW - 2), v2.dtype)], axis=1)


def _assemble_body(swarm_ref, ob0_ref, ob1_ref, rv_ref, nidb_ref, ndb_ref,
                   svec_ref, out_ref):
    sv = svec_ref[...]
    gdown = sv[:, 0:2]
    scale = sv[:, 2:3]
    ux = sv[:, 3:5]
    uy = sv[:, 5:7]
    out = gdown + swarm_ref[...]
    rv = rv_ref[...]
    ob_raw = ob0_ref[0][:, 0:2] + ob1_ref[0][:, 0:2]
    cap = jnp.abs(rv * scale)
    out = out - jnp.sign(ob_raw) * jnp.minimum(jnp.abs(ob_raw), cap)
    nidb = nidb_ref[...]
    ndb = ndb_ref[...]
    wall = jnp.zeros_like(out)
    for axis, u in ((0, ux), (1, uy)):
        col = slice(axis, axis + 1)
        active = nidb[:, col] > 1e-07
        neg = ndb[:, col] < 0
        v = jnp.clip(-rv[:, col], 0.0, 1000.0)
        contrib_neg = v * u
        contrib_pos = nidb[:, col] * v * u
        wall = wall + jnp.where(active & neg, contrib_neg, 0.0)
        wall = wall + jnp.where(active & (~neg), contrib_pos, 0.0)
    out_ref[...] = out - wall


def _full(shape):
    return pl.BlockSpec(shape, lambda *_: tuple(0 for _ in shape))


def _bd8(W):
    return block_diag(*([W] * _PK))


def kernel(x, edge_index, edge_attr, edge_index3, recent_velocity, unit_x,
           unit_y, down_direction, approach_speed, inv_edge_distance3,
           edge_direction3, norm_inv_distance_to_boundary,
           norm_distance_to_boundary, acceleration_scale, velocity_scale,
           params):
    N, E, E3, HID = _N, _E, _E3, _HID
    E8 = E // _PK
    W256 = HID * _PK

    pe_in = params['edge_in1']
    l1, l2 = params['layers1']
    pout = params['node_out1']

    def rt(b):
        return jnp.tile(b.reshape(1, -1), (1, _PK))

    Mavg = _bd8(jnp.full((HID, HID), 1.0 / HID, jnp.float32))

    ein = (_bd8(pe_in['Ws'][0]), rt(pe_in['bs'][0]), _bd8(pe_in['Ws'][1]),
           rt(pe_in['bs'][1]), _bd8(pe_in['Ws'][2]), rt(pe_in['bs'][2]),
           rt(pe_in['g']))

    def edge_layer_w(p):
        W = p['edge']['Ws'][0]
        D = _bd8(W[:HID] - W[HID:2 * HID])
        C = _bd8(W[2 * HID:])
        tail = (rt(p['edge']['bs'][0]), _bd8(p['edge']['Ws'][1]),
                rt(p['edge']['bs'][1]), _bd8(p['edge']['Ws'][2]),
                rt(p['edge']['bs'][2]), rt(p['edge']['g']))
        return D, C, tail

    def node_layer_w(p):
        W = p['node']['Ws'][0]
        P = _bd8(W[:HID])
        Q = _bd8(W[HID:])
        tail = (rt(p['node']['bs'][0]), _bd8(p['node']['Ws'][1]),
                rt(p['node']['bs'][1]), _bd8(p['node']['Ws'][2]),
                rt(p['node']['bs'][2]), rt(p['node']['g']))
        return P, Q, tail

    _, C1, etail1 = edge_layer_w(l1)
    D2, C2, etail2 = edge_layer_w(l2)
    _, Q1, ntail1 = node_layer_w(l1)
    P2, Q2, ntail2 = node_layer_w(l2)
    oW1 = _bd8(pout['Ws'][0])
    ob1w = rt(pout['bs'][0])
    oW2 = _bd8(pout['Ws'][1])

    g_const = 5.5339e-05 / acceleration_scale
    scale = velocity_scale / acceleration_scale * 60.0 / 94.0
    svec = jnp.concatenate([
        (g_const * down_direction).reshape(2),
        scale.reshape(1),
        (-unit_x * scale).reshape(2),
        (-unit_y * scale).reshape(2),
        (scale * 10.0).reshape(1),
    ]).reshape(1, 8)

    src, dst = edge_index[0], edge_index[1]
    zeros32 = jnp.zeros((_NP, HID), jnp.float32)
    zeros8 = jnp.zeros((_NP, _OBW), jnp.float32)

    EB8 = 2000
    grid = (E8 // EB8,)
    ea_p = edge_attr.reshape(E8, 14 * _PK)
    wspecs1 = [_full((W256, W256))] + [_full(w.shape) for w in ein] + \
              [_full(C1.shape)] + [_full(w.shape) for w in etail1]
    e1p, ef1p = pl.pallas_call(
        _edge_stage1_body,
        grid=grid,
        in_specs=[pl.BlockSpec((EB8, 14 * _PK), lambda i: (i, 0))] + wspecs1,
        out_specs=[pl.BlockSpec((EB8, W256), lambda i: (i, 0))] * 2,
        out_shape=[jax.ShapeDtypeStruct((E8, W256), jnp.float32)] * 2,
    )(ea_p, Mavg, *ein, C1, *etail1)

    aggr1 = _sc_segsum(e1p.reshape(E, HID), dst, zeros32)

    a1p = aggr1.reshape(_NC, _NP8, W256)
    wspecsn1 = [_full((W256, W256)), _full(Q1.shape)] + \
               [_full(w.shape) for w in ntail1] + [_full(D2.shape)]
    nf1p, nf1d = pl.pallas_call(
        _node_stage1_body,
        grid=(1,),
        in_specs=[pl.BlockSpec((1, _NP8, W256), lambda i: (0, 0, 0)),
                  pl.BlockSpec((1, _NP8, W256), lambda i: (1, 0, 0))] + wspecsn1,
        out_specs=[pl.BlockSpec((_NP8, W256), lambda i: (0, 0))] * 2,
        out_shape=[jax.ShapeDtypeStruct((_NP8, W256), jnp.float32)] * 2,
    )(a1p, a1p, Mavg, Q1, *ntail1, D2)

    td = _sc_gatherdiff(nf1d.reshape(_NP, HID), src, dst)

    wspecs2 = [_full((W256, W256)), _full(C2.shape)] + \
              [_full(w.shape) for w in etail2]
    e2p = pl.pallas_call(
        _edge_stage2_body,
        grid=grid,
        in_specs=[pl.BlockSpec((EB8, W256), lambda i: (i, 0))] * 2 + wspecs2,
        out_specs=pl.BlockSpec((EB8, W256), lambda i: (i, 0)),
        out_shape=jax.ShapeDtypeStruct((E8, W256), jnp.float32),
    )(td.reshape(E8, W256), ef1p, Mavg, C2, *etail2)

    aggr2 = _sc_segsum(e2p.reshape(E, HID), dst, zeros32)

    VB = 8192
    vgrid = (_E3P // VB,)
    pad3 = _E3P - E3
    inv_p = jnp.pad(inv_edge_distance3, ((0, pad3), (0, 0)))
    app_p = jnp.pad(approach_speed, ((0, pad3), (0, 0)))
    dir_p = jnp.pad(edge_direction3, ((0, pad3), (0, 0)))
    idx3_p = jnp.pad(edge_index3[0], ((0, pad3),))
    vals8 = pl.pallas_call(
        _vals3_body,
        grid=vgrid,
        in_specs=[pl.BlockSpec((VB, 1), lambda i: (i, 0)),
                  pl.BlockSpec((VB, 1), lambda i: (i, 0)),
                  pl.BlockSpec((VB, 2), lambda i: (i, 0)),
                  _full((1, 8))],
        out_specs=pl.BlockSpec((VB, _OBW), lambda i: (i, 0)),
        out_shape=jax.ShapeDtypeStruct((_E3P, _OBW), jnp.float32),
    )(inv_p, app_p, dir_p, svec)
    ob_part = _sc_obscatter(vals8, idx3_p, zeros8)

    a2p = aggr2.reshape(_NC, _NP8, W256)
    wspecsn2 = ([_full((W256, W256)), _full(P2.shape), _full(Q2.shape)] +
                [_full(w.shape) for w in ntail2] +
                [_full(oW1.shape), _full(ob1w.shape), _full(oW2.shape)])
    swarm = pl.pallas_call(
        _node_stage2_body,
        grid=(1,),
        in_specs=[pl.BlockSpec((_NP8, W256), lambda i: (0, 0)),
                  pl.BlockSpec((1, _NP8, W256), lambda i: (0, 0, 0)),
                  pl.BlockSpec((1, _NP8, W256), lambda i: (1, 0, 0))] + wspecsn2,
        out_specs=pl.BlockSpec((_NP8, 2 * _PK), lambda i: (0, 0)),
        out_shape=jax.ShapeDtypeStruct((_NP8, 2 * _PK), jnp.float32),
    )(nf1p, a2p, a2p, Mavg, P2, Q2, *ntail2, oW1, ob1w, oW2)

    NB = 2000
    ngrid = (N // NB,)
    out = pl.pallas_call(
        _assemble_body,
        grid=ngrid,
        in_specs=[pl.BlockSpec((NB, 2), lambda i: (i, 0)),
                  pl.BlockSpec((1, NB, _OBW), lambda i: (0, i, 0)),
                  pl.BlockSpec((1, NB, _OBW), lambda i: (1, i, 0))] +
                 [pl.BlockSpec((NB, 2), lambda i: (i, 0))] * 3 +
                 [_full((1, 8))],
        out_specs=pl.BlockSpec((NB, 2), lambda i: (i, 0)),
        out_shape=jax.ShapeDtypeStruct((N, 2), jnp.float32),
    )(swarm.reshape(_NP, 2), ob_part, ob_part, recent_velocity,
      norm_inv_distance_to_boundary, norm_distance_to_boundary, svec)

    return out


# speedup vs baseline: 4.9173x; 1.1076x over previous
import functools
import jax
import jax.numpy as jnp
from jax import lax
from jax.experimental import pallas as pl
from jax.experimental.pallas import tpu as pltpu
from jax.experimental.pallas import tpu_sc as plsc
from jax.scipy.linalg import block_diag

_EPS = 1e-5

_N = 10000
_E = 160000
_E3 = 40000
_E3P = 40960
_HID = 32
_PK = 8
_NC, _NS = 2, 16
_NW = _NC * _NS

_EPAD = 163840
_TILE_E = _EPAD // _NW
_CHUNK = 128
_GC = 4
_NG = _TILE_E // (_CHUNK * _GC)
_NB = 3
_GC3 = 5
_NG3 = 2
_NP = 10240
_NP8 = _NP // _PK
_NROWS = _NP // _NS
_TILE_E3 = _E3P // _NW
_NFULL3 = _TILE_E3 // _CHUNK
_OBW = 8

_MESH = functools.partial(plsc.VectorSubcoreMesh, core_axis_name="c",
                          subcore_axis_name="s", num_cores=_NC,
                          num_subcores=_NS)
_SC_PARAMS = pltpu.CompilerParams(use_tc_tiling_on_sc=False)


def _make_scatter_add_body(gc, ng, nb):
    def body(e_hbm, dst_hbm, zeros_hbm, out_hbm, idx_v, rows_v, acc_sh,
             sem_i, sem_r, sem_s):
        c = lax.axis_index("c")
        s = lax.axis_index("s")
        row0 = s * _NROWS
        pltpu.sync_copy(zeros_hbm.at[pl.ds(row0, _NROWS)],
                        acc_sh.at[pl.ds(row0, _NROWS)])
        plsc.subcore_barrier()
        cbase = (c * _NS + s) * ng * gc
        ldescs = [None] * ng
        sdescs = [None] * ng

        def scatters(g):
            b = g % nb
            i1, r1 = ldescs[g]
            i1.wait()
            r1.wait()
            ds = []
            for j in range(gc):
                ds.append(pltpu.async_copy(rows_v.at[b, j],
                                           acc_sh.at[idx_v.at[b, j]],
                                           sem_s, add=True))
            sdescs[g] = ds

        for g in range(ng):
            b = g % nb
            if g >= nb:
                for dsc in sdescs[g - nb]:
                    dsc.wait()
            off = cbase + g * gc
            ldescs[g] = (
                pltpu.async_copy(dst_hbm.at[pl.ds(off, gc)], idx_v.at[b],
                                 sem_i),
                pltpu.async_copy(e_hbm.at[pl.ds(off, gc)], rows_v.at[b],
                                 sem_r),
            )
            if g >= 1:
                scatters(g - 1)
        scatters(ng - 1)
        for g in range(max(0, ng - nb), ng):
            for dsc in sdescs[g]:
                dsc.wait()
        plsc.subcore_barrier()
        pltpu.sync_copy(acc_sh.at[pl.ds(row0, _NROWS)],
                        out_hbm.at[c, pl.ds(row0, _NROWS)])

    return body


def _sc_gatherdiff_body(tab_hbm, src_hbm, dst_hbm, d_hbm, idxs_v, idxd_v,
                        rs_v, rd_v, tab_sh, sem_i, sem_g, sem_w):
    c = lax.axis_index("c")
    s = lax.axis_index("s")
    row0 = s * _NROWS
    pltpu.sync_copy(tab_hbm.at[pl.ds(row0, _NROWS)],
                    tab_sh.at[pl.ds(row0, _NROWS)])
    plsc.subcore_barrier()
    cbase = (c * _NS + s) * _NG * _GC

    def group(g, _):
        off = cbase + g * _GC
        i1 = pltpu.async_copy(src_hbm.at[pl.ds(off, _GC)], idxs_v, sem_i)
        i2 = pltpu.async_copy(dst_hbm.at[pl.ds(off, _GC)], idxd_v, sem_i)
        i1.wait()
        i2.wait()
        gds = []
        for j in range(_GC):
            gds.append((
                pltpu.async_copy(tab_sh.at[idxd_v.at[j]], rd_v.at[j], sem_g),
                pltpu.async_copy(tab_sh.at[idxs_v.at[j]], rs_v.at[j], sem_g),
            ))
        for j in range(_GC):
            d1, d2 = gds[j]
            d1.wait()
            d2.wait()
            for r in range(_CHUNK):
                for k in range(_HID // 16):
                    sl = pl.ds(k * 16, 16)
                    rd_v[j, r, sl] = rd_v[j, r, sl] - rs_v[j, r, sl]
        pltpu.async_copy(rd_v, d_hbm.at[pl.ds(off, _GC)], sem_w).wait()
        return ()

    lax.fori_loop(0, _NG, group, (), unroll=False)


@functools.lru_cache(maxsize=None)
def _sc_kernels():
    mesh = _MESH()
    seg = pl.kernel(
        _make_scatter_add_body(_GC, _NG, _NB),
        compiler_params=_SC_PARAMS,
        out_type=jax.ShapeDtypeStruct((_NC, _NP, _HID), jnp.float32),
        mesh=mesh,
        scratch_types=[
            pltpu.VMEM((_NB, _GC, _CHUNK), jnp.int32),
            pltpu.VMEM((_NB, _GC, _CHUNK, _HID), jnp.float32),
            pltpu.VMEM_SHARED((_NP, _HID), jnp.float32),
            pltpu.SemaphoreType.DMA,
            pltpu.SemaphoreType.DMA,
            pltpu.SemaphoreType.DMA,
        ],
    )
    gd = pl.kernel(
        _sc_gatherdiff_body,
        compiler_params=_SC_PARAMS,
        out_type=jax.ShapeDtypeStruct((_EPAD // _CHUNK, _CHUNK, _HID),
                                      jnp.float32),
        mesh=mesh,
        scratch_types=[
            pltpu.VMEM((_GC, _CHUNK), jnp.int32),
            pltpu.VMEM((_GC, _CHUNK), jnp.int32),
            pltpu.VMEM((_GC, _CHUNK, _HID), jnp.float32),
            pltpu.VMEM((_GC, _CHUNK, _HID), jnp.float32),
            pltpu.VMEM_SHARED((_NP, _HID), jnp.float32),
            pltpu.SemaphoreType.DMA,
            pltpu.SemaphoreType.DMA,
            pltpu.SemaphoreType.DMA,
        ],
    )
    ob = pl.kernel(
        _make_scatter_add_body(_GC3, _NG3, _NB),
        compiler_params=_SC_PARAMS,
        out_type=jax.ShapeDtypeStruct((_NC, _NP, _OBW), jnp.float32),
        mesh=mesh,
        scratch_types=[
            pltpu.VMEM((_NB, _GC3, _CHUNK), jnp.int32),
            pltpu.VMEM((_NB, _GC3, _CHUNK, _OBW), jnp.float32),
            pltpu.VMEM_SHARED((_NP, _OBW), jnp.float32),
            pltpu.SemaphoreType.DMA,
            pltpu.SemaphoreType.DMA,
            pltpu.SemaphoreType.DMA,
        ],
    )
    return seg, gd, ob


def _sc_segsum(e3, dst2, zeros):
    return _sc_kernels()[0](e3, dst2, zeros)


def _sc_gatherdiff(tab, src2, dst2):
    return _sc_kernels()[1](tab, src2, dst2)


def _sc_obscatter(vals3, idx2, zeros):
    return _sc_kernels()[2](vals3, idx2, zeros)


def _dot(a, b):
    return jnp.dot(a, b, preferred_element_type=jnp.float32)


def _antisym_tail_p(t, b1, W2, b2, W3, b3, g, M):
    m = t.shape[0]
    tt = jnp.concatenate([t, -t], axis=0)
    a = jnp.maximum(tt + b1, 0.0)
    a = jnp.maximum(_dot(a, W2) + b2, 0.0)
    a = _dot(a, W3) + b3
    if g is not None:
        mu = _dot(a, M)
        c = a - mu
        var = _dot(c * c, M)
        a = c * jax.lax.rsqrt(var + _EPS) * g
    return 0.5 * (a[:m] - a[m:])


def _edge_stage1_body(ea_ref, M,
                      eW1, eb1, eW2, eb2, eW3, eb3, eg,
                      C1, fb1, fW2, fb2, fW3, fb3, fg,
                      e1_ref, ef1_ref):
    t = _dot(ea_ref[...], eW1[...])
    ef = _antisym_tail_p(t, eb1[...], eW2[...], eb2[...], eW3[...], eb3[...],
                         eg[...], M[...])
    t2 = _dot(ef, C1[...])
    e1 = _antisym_tail_p(t2, fb1[...], fW2[...], fb2[...], fW3[...], fb3[...],
                         fg[...], M[...])
    e1_ref[...] = e1
    ef1_ref[...] = ef + e1


def _edge_stage2_body(td_ref, ef1_ref, M,
                      C2, fb1, fW2, fb2, fW3, fb3, fg,
                      e2_ref):
    t = td_ref[...] + _dot(ef1_ref[...], C2[...])
    e2_ref[...] = _antisym_tail_p(t, fb1[...], fW2[...], fb2[...], fW3[...],
                                  fb3[...], fg[...], M[...])


def _node_stage1_body(p0_ref, p1_ref, M, Q1, nb1, nW2, nb2, nW3, nb3, ng, D2,
                      nf1_ref, nf1d_ref):
    t = _dot(p0_ref[0] + p1_ref[0], Q1[...])
    nf1 = _antisym_tail_p(t, nb1[...], nW2[...], nb2[...], nW3[...], nb3[...],
                          ng[...], M[...])
    nf1_ref[...] = nf1
    nf1d_ref[...] = _dot(nf1, D2[...])


def _node_stage2_body(nf1_ref, a0_ref, a1_ref, M,
                      P2, Q2, nb1, nW2, nb2, nW3, nb3, ng,
                      oW1, ob1w, oW2,
                      swarm_ref):
    nf1 = nf1_ref[...]
    t = _dot(nf1, P2[...]) + _dot(a0_ref[0] + a1_ref[0], Q2[...])
    nf2 = nf1 + _antisym_tail_p(t, nb1[...], nW2[...], nb2[...], nW3[...],
                                nb3[...], ng[...], M[...])
    t3 = _dot(nf2, oW1[...])
    swarm_ref[...] = 0.5 * _dot(
        jnp.maximum(t3 + ob1w[...], 0.0) - jnp.maximum(-t3 + ob1w[...], 0.0),
        oW2[...])


def _vals3_body(inv_ref, app_ref, dir_ref, svec_ref, out_ref):
    s10 = svec_ref[:, 7:8]
    v2 = inv_ref[...] * app_ref[...] * dir_ref[...] * s10
    out_ref[...] = jnp.concatenate(
        [v2, jnp.zeros((v2.shape[0], _OBW - 2), v2.dtype)], axis=1)


def _assemble_body(swarm_ref, ob0_ref, ob1_ref, rv_ref, nidb_ref, ndb_ref,
                   svec_ref, out_ref):
    sv = svec_ref[...]
    gdown = sv[:, 0:2]
    scale = sv[:, 2:3]
    ux = sv[:, 3:5]
    uy = sv[:, 5:7]
    out = gdown + swarm_ref[...]
    rv = rv_ref[...]
    ob_raw = ob0_ref[0][:, 0:2] + ob1_ref[0][:, 0:2]
    cap = jnp.abs(rv * scale)
    out = out - jnp.sign(ob_raw) * jnp.minimum(jnp.abs(ob_raw), cap)
    nidb = nidb_ref[...]
    ndb = ndb_ref[...]
    wall = jnp.zeros_like(out)
    for axis, u in ((0, ux), (1, uy)):
        col = slice(axis, axis + 1)
        active = nidb[:, col] > 1e-07
        neg = ndb[:, col] < 0
        v = jnp.clip(-rv[:, col], 0.0, 1000.0)
        contrib_neg = v * u
        contrib_pos = nidb[:, col] * v * u
        wall = wall + jnp.where(active & neg, contrib_neg, 0.0)
        wall = wall + jnp.where(active & (~neg), contrib_pos, 0.0)
    out_ref[...] = out - wall


def _full(shape):
    return pl.BlockSpec(shape, lambda *_: tuple(0 for _ in shape))


def _bd8(W):
    return block_diag(*([W] * _PK))


def kernel(x, edge_index, edge_attr, edge_index3, recent_velocity, unit_x,
           unit_y, down_direction, approach_speed, inv_edge_distance3,
           edge_direction3, norm_inv_distance_to_boundary,
           norm_distance_to_boundary, acceleration_scale, velocity_scale,
           params):
    N, E, E3, HID = _N, _E, _E3, _HID
    E8 = E // _PK
    W256 = HID * _PK

    pe_in = params['edge_in1']
    l1, l2 = params['layers1']
    pout = params['node_out1']

    def rt(b):
        return jnp.tile(b.reshape(1, -1), (1, _PK))

    Mavg = _bd8(jnp.full((HID, HID), 1.0 / HID, jnp.float32))

    ein = (_bd8(pe_in['Ws'][0]), rt(pe_in['bs'][0]), _bd8(pe_in['Ws'][1]),
           rt(pe_in['bs'][1]), _bd8(pe_in['Ws'][2]), rt(pe_in['bs'][2]),
           rt(pe_in['g']))

    def edge_layer_w(p):
        W = p['edge']['Ws'][0]
        D = _bd8(W[:HID] - W[HID:2 * HID])
        C = _bd8(W[2 * HID:])
        tail = (rt(p['edge']['bs'][0]), _bd8(p['edge']['Ws'][1]),
                rt(p['edge']['bs'][1]), _bd8(p['edge']['Ws'][2]),
                rt(p['edge']['bs'][2]), rt(p['edge']['g']))
        return D, C, tail

    def node_layer_w(p):
        W = p['node']['Ws'][0]
        P = _bd8(W[:HID])
        Q = _bd8(W[HID:])
        tail = (rt(p['node']['bs'][0]), _bd8(p['node']['Ws'][1]),
                rt(p['node']['bs'][1]), _bd8(p['node']['Ws'][2]),
                rt(p['node']['bs'][2]), rt(p['node']['g']))
        return P, Q, tail

    _, C1, etail1 = edge_layer_w(l1)
    D2, C2, etail2 = edge_layer_w(l2)
    _, Q1, ntail1 = node_layer_w(l1)
    P2, Q2, ntail2 = node_layer_w(l2)
    oW1 = _bd8(pout['Ws'][0])
    ob1w = rt(pout['bs'][0])
    oW2 = _bd8(pout['Ws'][1])

    g_const = 5.5339e-05 / acceleration_scale
    scale = velocity_scale / acceleration_scale * 60.0 / 94.0
    svec = jnp.concatenate([
        (g_const * down_direction).reshape(2),
        scale.reshape(1),
        (-unit_x * scale).reshape(2),
        (-unit_y * scale).reshape(2),
        (scale * 10.0).reshape(1),
    ]).reshape(1, 8)

    src, dst = edge_index[0], edge_index[1]
    zeros32 = jnp.zeros((_NP, HID), jnp.float32)
    zeros8 = jnp.zeros((_NP, _OBW), jnp.float32)

    E8P = _EPAD // _PK
    EB8 = 2048
    grid = (E8P // EB8,)
    ea_p = jnp.pad(edge_attr, ((0, _EPAD - E), (0, 0))).reshape(E8P, 14 * _PK)
    wspecs1 = [_full((W256, W256))] + [_full(w.shape) for w in ein] + \
              [_full(C1.shape)] + [_full(w.shape) for w in etail1]
    e1p, ef1p = pl.pallas_call(
        _edge_stage1_body,
        grid=grid,
        in_specs=[pl.BlockSpec((EB8, 14 * _PK), lambda i: (i, 0))] + wspecs1,
        out_specs=[pl.BlockSpec((EB8, W256), lambda i: (i, 0))] * 2,
        out_shape=[jax.ShapeDtypeStruct((E8P, W256), jnp.float32)] * 2,
    )(ea_p, Mavg, *ein, C1, *etail1)

    NCH = _EPAD // _CHUNK
    srcp = jnp.pad(src, (0, _EPAD - E)).reshape(NCH, _CHUNK)
    dstp = jnp.pad(dst, (0, _EPAD - E)).reshape(NCH, _CHUNK)
    aggr1 = _sc_segsum(e1p.reshape(NCH, _CHUNK, HID), dstp, zeros32)

    a1p = aggr1.reshape(_NC, _NP8, W256)
    wspecsn1 = [_full((W256, W256)), _full(Q1.shape)] + \
               [_full(w.shape) for w in ntail1] + [_full(D2.shape)]
    nf1p, nf1d = pl.pallas_call(
        _node_stage1_body,
        grid=(1,),
        in_specs=[pl.BlockSpec((1, _NP8, W256), lambda i: (0, 0, 0)),
                  pl.BlockSpec((1, _NP8, W256), lambda i: (1, 0, 0))] + wspecsn1,
        out_specs=[pl.BlockSpec((_NP8, W256), lambda i: (0, 0))] * 2,
        out_shape=[jax.ShapeDtypeStruct((_NP8, W256), jnp.float32)] * 2,
    )(a1p, a1p, Mavg, Q1, *ntail1, D2)

    td = _sc_gatherdiff(nf1d.reshape(_NP, HID), srcp, dstp)

    wspecs2 = [_full((W256, W256)), _full(C2.shape)] + \
              [_full(w.shape) for w in etail2]
    e2p = pl.pallas_call(
        _edge_stage2_body,
        grid=grid,
        in_specs=[pl.BlockSpec((EB8, W256), lambda i: (i, 0))] * 2 + wspecs2,
        out_specs=pl.BlockSpec((EB8, W256), lambda i: (i, 0)),
        out_shape=jax.ShapeDtypeStruct((E8P, W256), jnp.float32),
    )(td.reshape(E8P, W256), ef1p, Mavg, C2, *etail2)

    aggr2 = _sc_segsum(e2p.reshape(NCH, _CHUNK, HID), dstp, zeros32)

    VB = 8192
    vgrid = (_E3P // VB,)
    pad3 = _E3P - E3
    inv_p = jnp.pad(inv_edge_distance3, ((0, pad3), (0, 0)))
    app_p = jnp.pad(approach_speed, ((0, pad3), (0, 0)))
    dir_p = jnp.pad(edge_direction3, ((0, pad3), (0, 0)))
    idx3_p = jnp.pad(edge_index3[0], ((0, pad3),))
    vals8 = pl.pallas_call(
        _vals3_body,
        grid=vgrid,
        in_specs=[pl.BlockSpec((VB, 1), lambda i: (i, 0)),
                  pl.BlockSpec((VB, 1), lambda i: (i, 0)),
                  pl.BlockSpec((VB, 2), lambda i: (i, 0)),
                  _full((1, 8))],
        out_specs=pl.BlockSpec((VB, _OBW), lambda i: (i, 0)),
        out_shape=jax.ShapeDtypeStruct((_E3P, _OBW), jnp.float32),
    )(inv_p, app_p, dir_p, svec)
    ob_part = _sc_obscatter(vals8.reshape(_E3P // _CHUNK, _CHUNK, _OBW),
                            idx3_p.reshape(_E3P // _CHUNK, _CHUNK), zeros8)

    a2p = aggr2.reshape(_NC, _NP8, W256)
    wspecsn2 = ([_full((W256, W256)), _full(P2.shape), _full(Q2.shape)] +
                [_full(w.shape) for w in ntail2] +
                [_full(oW1.shape), _full(ob1w.shape), _full(oW2.shape)])
    swarm = pl.pallas_call(
        _node_stage2_body,
        grid=(1,),
        in_specs=[pl.BlockSpec((_NP8, W256), lambda i: (0, 0)),
                  pl.BlockSpec((1, _NP8, W256), lambda i: (0, 0, 0)),
                  pl.BlockSpec((1, _NP8, W256), lambda i: (1, 0, 0))] + wspecsn2,
        out_specs=pl.BlockSpec((_NP8, 2 * _PK), lambda i: (0, 0)),
        out_shape=jax.ShapeDtypeStruct((_NP8, 2 * _PK), jnp.float32),
    )(nf1p, a2p, a2p, Mavg, P2, Q2, *ntail2, oW1, ob1w, oW2)

    NB = 2000
    ngrid = (N // NB,)
    out = pl.pallas_call(
        _assemble_body,
        grid=ngrid,
        in_specs=[pl.BlockSpec((NB, 2), lambda i: (i, 0)),
                  pl.BlockSpec((1, NB, _OBW), lambda i: (0, i, 0)),
                  pl.BlockSpec((1, NB, _OBW), lambda i: (1, i, 0))] +
                 [pl.BlockSpec((NB, 2), lambda i: (i, 0))] * 3 +
                 [_full((1, 8))],
        out_specs=pl.BlockSpec((NB, 2), lambda i: (i, 0)),
        out_shape=jax.ShapeDtypeStruct((N, 2), jnp.float32),
    )(swarm.reshape(_NP, 2), ob_part, ob_part, recent_velocity,
      norm_inv_distance_to_boundary, norm_distance_to_boundary, svec)

    return out


# speedup vs baseline: 5.3992x; 1.0980x over previous
import functools
import jax
import jax.numpy as jnp
from jax import lax
from jax.experimental import pallas as pl
from jax.experimental.pallas import tpu as pltpu
from jax.experimental.pallas import tpu_sc as plsc
from jax.scipy.linalg import block_diag

_EPS = 1e-5

_N = 10000
_E = 160000
_E3 = 40000
_E3P = 40960
_HID = 32
_PK = 8
_NC, _NS = 2, 16
_NW = _NC * _NS

_EPAD = 163840
_TILE_E = _EPAD // _NW
_CHUNK = 128
_GC = 4
_NG = _TILE_E // (_CHUNK * _GC)
_NB = 3
_GC3 = 5
_NG3 = 2
_NP = 10240
_NP8 = _NP // _PK
_NROWS = _NP // _NS
_TILE_E3 = _E3P // _NW
_NFULL3 = _TILE_E3 // _CHUNK
_OBW = 8

_MESH = functools.partial(plsc.VectorSubcoreMesh, core_axis_name="c",
                          subcore_axis_name="s", num_cores=_NC,
                          num_subcores=_NS)
_SC_PARAMS = pltpu.CompilerParams(use_tc_tiling_on_sc=False)


def _make_scatter_add_body(gc, ng, nb):
    def body(e_hbm, dst_hbm, zeros_hbm, out_hbm, idx_v, rows_v, acc_sh,
             sem_i, sem_r, sem_s):
        c = lax.axis_index("c")
        s = lax.axis_index("s")
        row0 = s * _NROWS
        pltpu.sync_copy(zeros_hbm.at[pl.ds(row0, _NROWS)],
                        acc_sh.at[pl.ds(row0, _NROWS)])
        plsc.subcore_barrier()
        cbase = (c * _NS + s) * ng * gc
        ldescs = [None] * ng
        sdescs = [None] * ng

        def scatters(g):
            b = g % nb
            i1, r1 = ldescs[g]
            i1.wait()
            r1.wait()
            ds = []
            for j in range(gc):
                ds.append(pltpu.async_copy(rows_v.at[b, j],
                                           acc_sh.at[idx_v.at[b, j]],
                                           sem_s, add=True))
            sdescs[g] = ds

        for g in range(ng):
            b = g % nb
            if g >= nb:
                for dsc in sdescs[g - nb]:
                    dsc.wait()
            off = cbase + g * gc
            ldescs[g] = (
                pltpu.async_copy(dst_hbm.at[pl.ds(off, gc)], idx_v.at[b],
                                 sem_i),
                pltpu.async_copy(e_hbm.at[pl.ds(off, gc)], rows_v.at[b],
                                 sem_r),
            )
            if g >= 1:
                scatters(g - 1)
        scatters(ng - 1)
        for g in range(max(0, ng - nb), ng):
            for dsc in sdescs[g]:
                dsc.wait()
        plsc.subcore_barrier()
        pltpu.sync_copy(acc_sh.at[pl.ds(row0, _NROWS)],
                        out_hbm.at[c, pl.ds(row0, _NROWS)])

    return body


def _sc_gatherdiff_body(tab_hbm, src_hbm, dst_hbm, d_hbm, idxs_v, idxd_v,
                        rs_v, rd_v, tab_sh, sem_i, sem_g, sem_w):
    c = lax.axis_index("c")
    s = lax.axis_index("s")
    row0 = s * _NROWS
    pltpu.sync_copy(tab_hbm.at[pl.ds(row0, _NROWS)],
                    tab_sh.at[pl.ds(row0, _NROWS)])
    plsc.subcore_barrier()
    cbase = (c * _NS + s) * _NG * _GC

    def group(g, _):
        off = cbase + g * _GC
        i1 = pltpu.async_copy(src_hbm.at[pl.ds(off, _GC)], idxs_v, sem_i)
        i2 = pltpu.async_copy(dst_hbm.at[pl.ds(off, _GC)], idxd_v, sem_i)
        i1.wait()
        i2.wait()
        gds = []
        for j in range(_GC):
            gds.append((
                pltpu.async_copy(tab_sh.at[idxd_v.at[j]], rd_v.at[j], sem_g),
                pltpu.async_copy(tab_sh.at[idxs_v.at[j]], rs_v.at[j], sem_g),
            ))
        for j in range(_GC):
            d1, d2 = gds[j]
            d1.wait()
            d2.wait()
            for r in range(_CHUNK):
                for k in range(_HID // 16):
                    sl = pl.ds(k * 16, 16)
                    rd_v[j, r, sl] = rd_v[j, r, sl] - rs_v[j, r, sl]
        pltpu.async_copy(rd_v, d_hbm.at[pl.ds(off, _GC)], sem_w).wait()
        return ()

    lax.fori_loop(0, _NG, group, (), unroll=False)


@functools.lru_cache(maxsize=None)
def _sc_kernels():
    mesh = _MESH()
    seg = pl.kernel(
        _make_scatter_add_body(_GC, _NG, _NB),
        compiler_params=_SC_PARAMS,
        out_type=jax.ShapeDtypeStruct((_NC, _NP, _HID), jnp.float32),
        mesh=mesh,
        scratch_types=[
            pltpu.VMEM((_NB, _GC, _CHUNK), jnp.int32),
            pltpu.VMEM((_NB, _GC, _CHUNK, _HID), jnp.float32),
            pltpu.VMEM_SHARED((_NP, _HID), jnp.float32),
            pltpu.SemaphoreType.DMA,
            pltpu.SemaphoreType.DMA,
            pltpu.SemaphoreType.DMA,
        ],
    )
    gd = pl.kernel(
        _sc_gatherdiff_body,
        compiler_params=_SC_PARAMS,
        out_type=jax.ShapeDtypeStruct((_EPAD // _CHUNK, _CHUNK, _HID),
                                      jnp.float32),
        mesh=mesh,
        scratch_types=[
            pltpu.VMEM((_GC, _CHUNK), jnp.int32),
            pltpu.VMEM((_GC, _CHUNK), jnp.int32),
            pltpu.VMEM((_GC, _CHUNK, _HID), jnp.float32),
            pltpu.VMEM((_GC, _CHUNK, _HID), jnp.float32),
            pltpu.VMEM_SHARED((_NP, _HID), jnp.float32),
            pltpu.SemaphoreType.DMA,
            pltpu.SemaphoreType.DMA,
            pltpu.SemaphoreType.DMA,
        ],
    )
    ob = pl.kernel(
        _make_scatter_add_body(_GC3, _NG3, _NB),
        compiler_params=_SC_PARAMS,
        out_type=jax.ShapeDtypeStruct((_NC, _NP, _OBW), jnp.float32),
        mesh=mesh,
        scratch_types=[
            pltpu.VMEM((_NB, _GC3, _CHUNK), jnp.int32),
            pltpu.VMEM((_NB, _GC3, _CHUNK, _OBW), jnp.float32),
            pltpu.VMEM_SHARED((_NP, _OBW), jnp.float32),
            pltpu.SemaphoreType.DMA,
            pltpu.SemaphoreType.DMA,
            pltpu.SemaphoreType.DMA,
        ],
    )
    return seg, gd, ob


def _sc_segsum(e3, dst2, zeros):
    return _sc_kernels()[0](e3, dst2, zeros)


def _sc_gatherdiff(tab, src2, dst2):
    return _sc_kernels()[1](tab, src2, dst2)


def _sc_obscatter(vals3, idx2, zeros):
    return _sc_kernels()[2](vals3, idx2, zeros)


def _dot(a, b):
    return jnp.dot(a, b, preferred_element_type=jnp.float32)


def _antisym_tail_p(t, b1, W2, b2, W3, b3, g, M):
    m = t.shape[0]
    tt = jnp.concatenate([t, -t], axis=0)
    a = jnp.maximum(tt + b1, 0.0)
    a = jnp.maximum(_dot(a, W2) + b2, 0.0)
    a = _dot(a, W3) + b3
    if g is not None:
        mu = _dot(a, M)
        c = a - mu
        var = _dot(c * c, M)
        a = c * jax.lax.rsqrt(var + _EPS) * g
    return 0.5 * (a[:m] - a[m:])


def _edge_stage1_body(ea_ref, M,
                      eW1, eb1, eW2, eb2, eW3, eb3, eg,
                      C1, fb1, fW2, fb2, fW3, fb3, fg,
                      e1_ref, ef1_ref):
    t = _dot(ea_ref[...], eW1[...])
    ef = _antisym_tail_p(t, eb1[...], eW2[...], eb2[...], eW3[...], eb3[...],
                         eg[...], M[...])
    t2 = _dot(ef, C1[...])
    e1 = _antisym_tail_p(t2, fb1[...], fW2[...], fb2[...], fW3[...], fb3[...],
                         fg[...], M[...])
    e1_ref[...] = e1
    ef1_ref[...] = ef + e1


def _edge_stage2_body(td_ref, ef1_ref, M,
                      C2, fb1, fW2, fb2, fW3, fb3, fg,
                      e2_ref):
    t = td_ref[...] + _dot(ef1_ref[...], C2[...])
    e2_ref[...] = _antisym_tail_p(t, fb1[...], fW2[...], fb2[...], fW3[...],
                                  fb3[...], fg[...], M[...])


def _node_stage1_body(p0_ref, p1_ref, M, Q1, nb1, nW2, nb2, nW3, nb3, ng, D2,
                      nf1_ref, nf1d_ref):
    t = _dot(p0_ref[0] + p1_ref[0], Q1[...])
    nf1 = _antisym_tail_p(t, nb1[...], nW2[...], nb2[...], nW3[...], nb3[...],
                          ng[...], M[...])
    nf1_ref[...] = nf1
    nf1d_ref[...] = _dot(nf1, D2[...])


def _node_stage2_body(nf1_ref, a0_ref, a1_ref, M,
                      P2, Q2, nb1, nW2, nb2, nW3, nb3, ng,
                      oW1, ob1w, oW2,
                      swarm_ref):
    nf1 = nf1_ref[...]
    t = _dot(nf1, P2[...]) + _dot(a0_ref[0] + a1_ref[0], Q2[...])
    nf2 = nf1 + _antisym_tail_p(t, nb1[...], nW2[...], nb2[...], nW3[...],
                                nb3[...], ng[...], M[...])
    t3 = _dot(nf2, oW1[...])
    swarm_ref[...] = 0.5 * _dot(
        jnp.maximum(t3 + ob1w[...], 0.0) - jnp.maximum(-t3 + ob1w[...], 0.0),
        oW2[...])


def _vals3_body(inv_ref, app_ref, dir_ref, svec_ref, out_ref):
    s10 = svec_ref[:, 7:8]
    v2 = inv_ref[...] * app_ref[...] * dir_ref[...] * s10
    out_ref[...] = jnp.concatenate(
        [v2, jnp.zeros((v2.shape[0], _OBW - 2), v2.dtype)], axis=1)


def _assemble_body(swarm_ref, ob0_ref, ob1_ref, rv_ref, nidb_ref, ndb_ref,
                   svec_ref, out_ref):
    sv = svec_ref[...]
    gdown = sv[:, 0:2]
    scale = sv[:, 2:3]
    ux = sv[:, 3:5]
    uy = sv[:, 5:7]
    out = gdown + swarm_ref[...]
    rv = rv_ref[...]
    ob_raw = ob0_ref[0][:, 0:2] + ob1_ref[0][:, 0:2]
    cap = jnp.abs(rv * scale)
    out = out - jnp.sign(ob_raw) * jnp.minimum(jnp.abs(ob_raw), cap)
    nidb = nidb_ref[...]
    ndb = ndb_ref[...]
    wall = jnp.zeros_like(out)
    for axis, u in ((0, ux), (1, uy)):
        col = slice(axis, axis + 1)
        active = nidb[:, col] > 1e-07
        neg = ndb[:, col] < 0
        v = jnp.clip(-rv[:, col], 0.0, 1000.0)
        contrib_neg = v * u
        contrib_pos = nidb[:, col] * v * u
        wall = wall + jnp.where(active & neg, contrib_neg, 0.0)
        wall = wall + jnp.where(active & (~neg), contrib_pos, 0.0)
    out_ref[...] = out - wall


def _full(shape):
    return pl.BlockSpec(shape, lambda *_: tuple(0 for _ in shape))


def _bd8(W):
    return block_diag(*([W] * _PK))


def kernel(x, edge_index, edge_attr, edge_index3, recent_velocity, unit_x,
           unit_y, down_direction, approach_speed, inv_edge_distance3,
           edge_direction3, norm_inv_distance_to_boundary,
           norm_distance_to_boundary, acceleration_scale, velocity_scale,
           params):
    N, E, E3, HID = _N, _E, _E3, _HID
    E8 = E // _PK
    W256 = HID * _PK

    pe_in = params['edge_in1']
    l1, l2 = params['layers1']
    pout = params['node_out1']

    def rt(b):
        return jnp.tile(b.reshape(1, -1), (1, _PK))

    Mavg = _bd8(jnp.full((HID, HID), 1.0 / HID, jnp.float32))

    ein = (_bd8(pe_in['Ws'][0]), rt(pe_in['bs'][0]), _bd8(pe_in['Ws'][1]),
           rt(pe_in['bs'][1]), _bd8(pe_in['Ws'][2]), rt(pe_in['bs'][2]),
           rt(pe_in['g']))

    def edge_layer_w(p):
        W = p['edge']['Ws'][0]
        D = _bd8(W[:HID] - W[HID:2 * HID])
        C = _bd8(W[2 * HID:])
        tail = (rt(p['edge']['bs'][0]), _bd8(p['edge']['Ws'][1]),
                rt(p['edge']['bs'][1]), _bd8(p['edge']['Ws'][2]),
                rt(p['edge']['bs'][2]), rt(p['edge']['g']))
        return D, C, tail

    def node_layer_w(p):
        W = p['node']['Ws'][0]
        P = _bd8(W[:HID])
        Q = _bd8(W[HID:])
        tail = (rt(p['node']['bs'][0]), _bd8(p['node']['Ws'][1]),
                rt(p['node']['bs'][1]), _bd8(p['node']['Ws'][2]),
                rt(p['node']['bs'][2]), rt(p['node']['g']))
        return P, Q, tail

    _, C1, etail1 = edge_layer_w(l1)
    D2, C2, etail2 = edge_layer_w(l2)
    _, Q1, ntail1 = node_layer_w(l1)
    P2, Q2, ntail2 = node_layer_w(l2)
    oW1 = _bd8(pout['Ws'][0])
    ob1w = rt(pout['bs'][0])
    oW2 = _bd8(pout['Ws'][1])

    g_const = 5.5339e-05 / acceleration_scale
    scale = velocity_scale / acceleration_scale * 60.0 / 94.0
    svec = jnp.concatenate([
        (g_const * down_direction).reshape(2),
        scale.reshape(1),
        (-unit_x * scale).reshape(2),
        (-unit_y * scale).reshape(2),
        (scale * 10.0).reshape(1),
    ]).reshape(1, 8)

    src, dst = edge_index[0], edge_index[1]
    zeros32 = jnp.zeros((_NP, HID), jnp.float32)
    zeros8 = jnp.zeros((_NP, _OBW), jnp.float32)

    E8P = _EPAD // _PK
    EB8 = 2000
    grid = (E8 // EB8,)
    grid2 = (E8P // 2048,)
    ea_p = edge_attr.reshape(E8, 14 * _PK)
    wspecs1 = [_full((W256, W256))] + [_full(w.shape) for w in ein] + \
              [_full(C1.shape)] + [_full(w.shape) for w in etail1]
    e1p, ef1p = pl.pallas_call(
        _edge_stage1_body,
        grid=grid,
        in_specs=[pl.BlockSpec((EB8, 14 * _PK), lambda i: (i, 0))] + wspecs1,
        out_specs=[pl.BlockSpec((EB8, W256), lambda i: (i, 0))] * 2,
        out_shape=[jax.ShapeDtypeStruct((E8P, W256), jnp.float32)] * 2,
    )(ea_p, Mavg, *ein, C1, *etail1)

    NCH = _EPAD // _CHUNK
    srcp = jnp.pad(src, (0, _EPAD - E), constant_values=_N).reshape(NCH, _CHUNK)
    dstp = jnp.pad(dst, (0, _EPAD - E), constant_values=_N).reshape(NCH, _CHUNK)
    aggr1 = _sc_segsum(e1p.reshape(NCH, _CHUNK, HID), dstp, zeros32)

    a1p = aggr1.reshape(_NC, _NP8, W256)
    wspecsn1 = [_full((W256, W256)), _full(Q1.shape)] + \
               [_full(w.shape) for w in ntail1] + [_full(D2.shape)]
    nf1p, nf1d = pl.pallas_call(
        _node_stage1_body,
        grid=(1,),
        in_specs=[pl.BlockSpec((1, _NP8, W256), lambda i: (0, 0, 0)),
                  pl.BlockSpec((1, _NP8, W256), lambda i: (1, 0, 0))] + wspecsn1,
        out_specs=[pl.BlockSpec((_NP8, W256), lambda i: (0, 0))] * 2,
        out_shape=[jax.ShapeDtypeStruct((_NP8, W256), jnp.float32)] * 2,
    )(a1p, a1p, Mavg, Q1, *ntail1, D2)

    td = _sc_gatherdiff(nf1d.reshape(_NP, HID), srcp, dstp)

    wspecs2 = [_full((W256, W256)), _full(C2.shape)] + \
              [_full(w.shape) for w in etail2]
    e2p = pl.pallas_call(
        _edge_stage2_body,
        grid=grid2,
        in_specs=[pl.BlockSpec((2048, W256), lambda i: (i, 0))] * 2 + wspecs2,
        out_specs=pl.BlockSpec((2048, W256), lambda i: (i, 0)),
        out_shape=jax.ShapeDtypeStruct((E8P, W256), jnp.float32),
    )(td.reshape(E8P, W256), ef1p, Mavg, C2, *etail2)

    aggr2 = _sc_segsum(e2p.reshape(NCH, _CHUNK, HID), dstp, zeros32)

    VB = 8192
    vgrid = (_E3P // VB,)
    pad3 = _E3P - E3
    inv_p = jnp.pad(inv_edge_distance3, ((0, pad3), (0, 0)))
    app_p = jnp.pad(approach_speed, ((0, pad3), (0, 0)))
    dir_p = jnp.pad(edge_direction3, ((0, pad3), (0, 0)))
    idx3_p = jnp.pad(edge_index3[0], ((0, pad3),))
    vals8 = pl.pallas_call(
        _vals3_body,
        grid=vgrid,
        in_specs=[pl.BlockSpec((VB, 1), lambda i: (i, 0)),
                  pl.BlockSpec((VB, 1), lambda i: (i, 0)),
                  pl.BlockSpec((VB, 2), lambda i: (i, 0)),
                  _full((1, 8))],
        out_specs=pl.BlockSpec((VB, _OBW), lambda i: (i, 0)),
        out_shape=jax.ShapeDtypeStruct((_E3P, _OBW), jnp.float32),
    )(inv_p, app_p, dir_p, svec)
    ob_part = _sc_obscatter(vals8.reshape(_E3P // _CHUNK, _CHUNK, _OBW),
                            idx3_p.reshape(_E3P // _CHUNK, _CHUNK), zeros8)

    a2p = aggr2.reshape(_NC, _NP8, W256)
    wspecsn2 = ([_full((W256, W256)), _full(P2.shape), _full(Q2.shape)] +
                [_full(w.shape) for w in ntail2] +
                [_full(oW1.shape), _full(ob1w.shape), _full(oW2.shape)])
    swarm = pl.pallas_call(
        _node_stage2_body,
        grid=(1,),
        in_specs=[pl.BlockSpec((_NP8, W256), lambda i: (0, 0)),
                  pl.BlockSpec((1, _NP8, W256), lambda i: (0, 0, 0)),
                  pl.BlockSpec((1, _NP8, W256), lambda i: (1, 0, 0))] + wspecsn2,
        out_specs=pl.BlockSpec((_NP8, 2 * _PK), lambda i: (0, 0)),
        out_shape=jax.ShapeDtypeStruct((_NP8, 2 * _PK), jnp.float32),
    )(nf1p, a2p, a2p, Mavg, P2, Q2, *ntail2, oW1, ob1w, oW2)

    NB = 2000
    ngrid = (N // NB,)
    out = pl.pallas_call(
        _assemble_body,
        grid=ngrid,
        in_specs=[pl.BlockSpec((NB, 2), lambda i: (i, 0)),
                  pl.BlockSpec((1, NB, _OBW), lambda i: (0, i, 0)),
                  pl.BlockSpec((1, NB, _OBW), lambda i: (1, i, 0))] +
                 [pl.BlockSpec((NB, 2), lambda i: (i, 0))] * 3 +
                 [_full((1, 8))],
        out_specs=pl.BlockSpec((NB, 2), lambda i: (i, 0)),
        out_shape=jax.ShapeDtypeStruct((N, 2), jnp.float32),
    )(swarm.reshape(_NP, 2), ob_part, ob_part, recent_velocity,
      norm_inv_distance_to_boundary, norm_distance_to_boundary, svec)

    return out


# speedup vs baseline: 5.4119x; 1.0024x over previous
import functools
import jax
import jax.numpy as jnp
from jax import lax
from jax.experimental import pallas as pl
from jax.experimental.pallas import tpu as pltpu
from jax.experimental.pallas import tpu_sc as plsc
from jax.scipy.linalg import block_diag

_EPS = 1e-5

_N = 10000
_E = 160000
_E3 = 40000
_E3P = 40960
_HID = 32
_PK = 8
_NC, _NS = 2, 16
_NW = _NC * _NS

_EPAD = 163840
_TILE_E = _EPAD // _NW
_CHUNK = 128
_GC = 8
_NG = _TILE_E // (_CHUNK * _GC)
_NB = 2
_GCG = 5
_NGG = _TILE_E // (_CHUNK * _GCG)
_ECH = _E // _CHUNK
_GC3 = 5
_NG3 = 2
_NP = 10240
_NP8 = _NP // _PK
_NROWS = _NP // _NS
_TILE_E3 = _E3P // _NW
_NFULL3 = _TILE_E3 // _CHUNK
_OBW = 8

_MESH = functools.partial(plsc.VectorSubcoreMesh, core_axis_name="c",
                          subcore_axis_name="s", num_cores=_NC,
                          num_subcores=_NS)
_SC_PARAMS = pltpu.CompilerParams(use_tc_tiling_on_sc=False)


def _make_scatter_add_body(gc, ng, nb):
    def body(e_hbm, dst_hbm, zeros_hbm, out_hbm, idx_v, rows_v, acc_sh,
             sem_i, sem_r, sem_s):
        c = lax.axis_index("c")
        s = lax.axis_index("s")
        row0 = s * _NROWS
        pltpu.sync_copy(zeros_hbm.at[pl.ds(row0, _NROWS)],
                        acc_sh.at[pl.ds(row0, _NROWS)])
        plsc.subcore_barrier()
        cbase = (c * _NS + s) * ng * gc
        ldescs = [None] * ng
        sdescs = [None] * ng

        def scatters(g):
            b = g % nb
            i1, r1 = ldescs[g]
            i1.wait()
            r1.wait()
            ds = []
            for j in range(gc):
                ds.append(pltpu.async_copy(rows_v.at[b, j],
                                           acc_sh.at[idx_v.at[b, j]],
                                           sem_s, add=True))
            sdescs[g] = ds

        for g in range(ng):
            b = g % nb
            if g >= nb:
                for dsc in sdescs[g - nb]:
                    dsc.wait()
            off = cbase + g * gc
            ldescs[g] = (
                pltpu.async_copy(dst_hbm.at[pl.ds(off, gc)], idx_v.at[b],
                                 sem_i),
                pltpu.async_copy(e_hbm.at[pl.ds(off, gc)], rows_v.at[b],
                                 sem_r),
            )
            if g >= 1:
                scatters(g - 1)
        scatters(ng - 1)
        for g in range(max(0, ng - nb), ng):
            for dsc in sdescs[g]:
                dsc.wait()
        plsc.subcore_barrier()
        pltpu.sync_copy(acc_sh.at[pl.ds(row0, _NROWS)],
                        out_hbm.at[c, pl.ds(row0, _NROWS)])

    return body


def _sc_gatherdiff_body(tab_hbm, src_hbm, dst_hbm, d_hbm, idxs_v, idxd_v,
                        rs_v, rd_v, tab_sh, sem_i, sem_g, sem_w):
    c = lax.axis_index("c")
    s = lax.axis_index("s")
    row0 = s * _NROWS
    pltpu.sync_copy(tab_hbm.at[pl.ds(row0, _NROWS)],
                    tab_sh.at[pl.ds(row0, _NROWS)])
    plsc.subcore_barrier()
    cbase = (c * _NS + s) * _NGG * _GCG

    def group(g, _):
        off = cbase + g * _GCG
        i1 = pltpu.async_copy(src_hbm.at[pl.ds(off, _GCG)], idxs_v, sem_i)
        i2 = pltpu.async_copy(dst_hbm.at[pl.ds(off, _GCG)], idxd_v, sem_i)
        i1.wait()
        i2.wait()
        gds = []
        for j in range(_GCG):
            gds.append((
                pltpu.async_copy(tab_sh.at[idxd_v.at[j]], rd_v.at[j], sem_g),
                pltpu.async_copy(tab_sh.at[idxs_v.at[j]], rs_v.at[j], sem_g),
            ))
        for j in range(_GCG):
            d1, d2 = gds[j]
            d1.wait()
            d2.wait()
            for r in range(_CHUNK):
                for k in range(_HID // 16):
                    sl = pl.ds(k * 16, 16)
                    rd_v[j, r, sl] = rd_v[j, r, sl] - rs_v[j, r, sl]
        pltpu.async_copy(rd_v, d_hbm.at[pl.ds(off, _GCG)], sem_w).wait()
        return ()

    lax.fori_loop(0, _NGG, group, (), unroll=False)


@functools.lru_cache(maxsize=None)
def _sc_kernels():
    mesh = _MESH()
    seg = pl.kernel(
        _make_scatter_add_body(_GC, _NG, _NB),
        compiler_params=_SC_PARAMS,
        out_type=jax.ShapeDtypeStruct((_NC, _NP, _HID), jnp.float32),
        mesh=mesh,
        scratch_types=[
            pltpu.VMEM((_NB, _GC, _CHUNK), jnp.int32),
            pltpu.VMEM((_NB, _GC, _CHUNK, _HID), jnp.float32),
            pltpu.VMEM_SHARED((_NP, _HID), jnp.float32),
            pltpu.SemaphoreType.DMA,
            pltpu.SemaphoreType.DMA,
            pltpu.SemaphoreType.DMA,
        ],
    )
    gd = pl.kernel(
        _sc_gatherdiff_body,
        compiler_params=_SC_PARAMS,
        out_type=jax.ShapeDtypeStruct((_EPAD // _CHUNK, _CHUNK, _HID),
                                      jnp.float32),
        mesh=mesh,
        scratch_types=[
            pltpu.VMEM((_GCG, _CHUNK), jnp.int32),
            pltpu.VMEM((_GCG, _CHUNK), jnp.int32),
            pltpu.VMEM((_GCG, _CHUNK, _HID), jnp.float32),
            pltpu.VMEM((_GCG, _CHUNK, _HID), jnp.float32),
            pltpu.VMEM_SHARED((_NP, _HID), jnp.float32),
            pltpu.SemaphoreType.DMA,
            pltpu.SemaphoreType.DMA,
            pltpu.SemaphoreType.DMA,
        ],
    )
    ob = pl.kernel(
        _make_scatter_add_body(_GC3, _NG3, _NB),
        compiler_params=_SC_PARAMS,
        out_type=jax.ShapeDtypeStruct((_NC, _NP, _OBW), jnp.float32),
        mesh=mesh,
        scratch_types=[
            pltpu.VMEM((_NB, _GC3, _CHUNK), jnp.int32),
            pltpu.VMEM((_NB, _GC3, _CHUNK, _OBW), jnp.float32),
            pltpu.VMEM_SHARED((_NP, _OBW), jnp.float32),
            pltpu.SemaphoreType.DMA,
            pltpu.SemaphoreType.DMA,
            pltpu.SemaphoreType.DMA,
        ],
    )
    return seg, gd, ob


def _sc_segsum(e3, dst2, zeros):
    return _sc_kernels()[0](e3, dst2, zeros)


def _sc_gatherdiff(tab, src2, dst2):
    return _sc_kernels()[1](tab, src2, dst2)


def _sc_obscatter(vals3, idx2, zeros):
    return _sc_kernels()[2](vals3, idx2, zeros)


def _dot(a, b):
    return jnp.dot(a, b, preferred_element_type=jnp.float32)


def _antisym_tail_p(t, b1, W2, b2, W3, b3, g, M):
    m = t.shape[0]
    tt = jnp.concatenate([t, -t], axis=0)
    a = jnp.maximum(tt + b1, 0.0)
    a = jnp.maximum(_dot(a, W2) + b2, 0.0)
    a = _dot(a, W3) + b3
    if g is not None:
        mu = _dot(a, M)
        c = a - mu
        var = _dot(c * c, M)
        a = c * jax.lax.rsqrt(var + _EPS) * g
    return 0.5 * (a[:m] - a[m:])


def _edge_stage1_body(ea_ref, M,
                      eW1, eb1, eW2, eb2, eW3, eb3, eg,
                      C1, fb1, fW2, fb2, fW3, fb3, fg,
                      e1_ref, ef1_ref):
    t = _dot(ea_ref[...], eW1[...])
    ef = _antisym_tail_p(t, eb1[...], eW2[...], eb2[...], eW3[...], eb3[...],
                         eg[...], M[...])
    t2 = _dot(ef, C1[...])
    e1 = _antisym_tail_p(t2, fb1[...], fW2[...], fb2[...], fW3[...], fb3[...],
                         fg[...], M[...])
    e1_ref[...] = e1
    ef1_ref[...] = ef + e1


def _edge_stage2_body(td_ref, ef1_ref, M,
                      C2, fb1, fW2, fb2, fW3, fb3, fg,
                      e2_ref):
    t = td_ref[...] + _dot(ef1_ref[...], C2[...])
    e2_ref[...] = _antisym_tail_p(t, fb1[...], fW2[...], fb2[...], fW3[...],
                                  fb3[...], fg[...], M[...])


def _node_stage1_body(p0_ref, p1_ref, M, Q1, nb1, nW2, nb2, nW3, nb3, ng, D2,
                      nf1_ref, nf1d_ref):
    t = _dot(p0_ref[0] + p1_ref[0], Q1[...])
    nf1 = _antisym_tail_p(t, nb1[...], nW2[...], nb2[...], nW3[...], nb3[...],
                          ng[...], M[...])
    nf1_ref[...] = nf1
    nf1d_ref[...] = _dot(nf1, D2[...])


def _node_stage2_body(nf1_ref, a0_ref, a1_ref, M,
                      P2, Q2, nb1, nW2, nb2, nW3, nb3, ng,
                      oW1, ob1w, oW2,
                      swarm_ref):
    nf1 = nf1_ref[...]
    t = _dot(nf1, P2[...]) + _dot(a0_ref[0] + a1_ref[0], Q2[...])
    nf2 = nf1 + _antisym_tail_p(t, nb1[...], nW2[...], nb2[...], nW3[...],
                                nb3[...], ng[...], M[...])
    t3 = _dot(nf2, oW1[...])
    swarm_ref[...] = 0.5 * _dot(
        jnp.maximum(t3 + ob1w[...], 0.0) - jnp.maximum(-t3 + ob1w[...], 0.0),
        oW2[...])


def _pad_idx_body(src_ref, dst_ref, srcp_ref, dstp_ref):
    i = pl.program_id(0)
    rowid = jax.lax.broadcasted_iota(jnp.int32, srcp_ref.shape, 0) + i * 128
    ok = rowid < _ECH
    srcp_ref[...] = jnp.where(ok, src_ref[...], _N)
    dstp_ref[...] = jnp.where(ok, dst_ref[...], _N)


def _vals3_body(inv_ref, app_ref, dir_ref, svec_ref, out_ref):
    s10 = svec_ref[:, 7:8]
    v2 = inv_ref[...] * app_ref[...] * dir_ref[...] * s10
    out_ref[...] = jnp.concatenate(
        [v2, jnp.zeros((v2.shape[0], _OBW - 2), v2.dtype)], axis=1)


def _assemble_body(swarm_ref, ob0_ref, ob1_ref, rv_ref, nidb_ref, ndb_ref,
                   svec_ref, out_ref):
    sv = svec_ref[...]
    gdown = sv[:, 0:2]
    scale = sv[:, 2:3]
    ux = sv[:, 3:5]
    uy = sv[:, 5:7]
    out = gdown + swarm_ref[...]
    rv = rv_ref[...]
    ob_raw = ob0_ref[0][:, 0:2] + ob1_ref[0][:, 0:2]
    cap = jnp.abs(rv * scale)
    out = out - jnp.sign(ob_raw) * jnp.minimum(jnp.abs(ob_raw), cap)
    nidb = nidb_ref[...]
    ndb = ndb_ref[...]
    wall = jnp.zeros_like(out)
    for axis, u in ((0, ux), (1, uy)):
        col = slice(axis, axis + 1)
        active = nidb[:, col] > 1e-07
        neg = ndb[:, col] < 0
        v = jnp.clip(-rv[:, col], 0.0, 1000.0)
        contrib_neg = v * u
        contrib_pos = nidb[:, col] * v * u
        wall = wall + jnp.where(active & neg, contrib_neg, 0.0)
        wall = wall + jnp.where(active & (~neg), contrib_pos, 0.0)
    out_ref[...] = out - wall


def _full(shape):
    return pl.BlockSpec(shape, lambda *_: tuple(0 for _ in shape))


def _bd8(W):
    return block_diag(*([W] * _PK))


def kernel(x, edge_index, edge_attr, edge_index3, recent_velocity, unit_x,
           unit_y, down_direction, approach_speed, inv_edge_distance3,
           edge_direction3, norm_inv_distance_to_boundary,
           norm_distance_to_boundary, acceleration_scale, velocity_scale,
           params):
    N, E, E3, HID = _N, _E, _E3, _HID
    E8 = E // _PK
    W256 = HID * _PK

    pe_in = params['edge_in1']
    l1, l2 = params['layers1']
    pout = params['node_out1']

    def rt(b):
        return jnp.tile(b.reshape(1, -1), (1, _PK))

    Mavg = _bd8(jnp.full((HID, HID), 1.0 / HID, jnp.float32))

    ein = (_bd8(pe_in['Ws'][0]), rt(pe_in['bs'][0]), _bd8(pe_in['Ws'][1]),
           rt(pe_in['bs'][1]), _bd8(pe_in['Ws'][2]), rt(pe_in['bs'][2]),
           rt(pe_in['g']))

    def edge_layer_w(p):
        W = p['edge']['Ws'][0]
        D = _bd8(W[:HID] - W[HID:2 * HID])
        C = _bd8(W[2 * HID:])
        tail = (rt(p['edge']['bs'][0]), _bd8(p['edge']['Ws'][1]),
                rt(p['edge']['bs'][1]), _bd8(p['edge']['Ws'][2]),
                rt(p['edge']['bs'][2]), rt(p['edge']['g']))
        return D, C, tail

    def node_layer_w(p):
        W = p['node']['Ws'][0]
        P = _bd8(W[:HID])
        Q = _bd8(W[HID:])
        tail = (rt(p['node']['bs'][0]), _bd8(p['node']['Ws'][1]),
                rt(p['node']['bs'][1]), _bd8(p['node']['Ws'][2]),
                rt(p['node']['bs'][2]), rt(p['node']['g']))
        return P, Q, tail

    _, C1, etail1 = edge_layer_w(l1)
    D2, C2, etail2 = edge_layer_w(l2)
    _, Q1, ntail1 = node_layer_w(l1)
    P2, Q2, ntail2 = node_layer_w(l2)
    oW1 = _bd8(pout['Ws'][0])
    ob1w = rt(pout['bs'][0])
    oW2 = _bd8(pout['Ws'][1])

    g_const = 5.5339e-05 / acceleration_scale
    scale = velocity_scale / acceleration_scale * 60.0 / 94.0
    svec = jnp.concatenate([
        (g_const * down_direction).reshape(2),
        scale.reshape(1),
        (-unit_x * scale).reshape(2),
        (-unit_y * scale).reshape(2),
        (scale * 10.0).reshape(1),
    ]).reshape(1, 8)

    src, dst = edge_index[0], edge_index[1]
    zeros32 = jnp.zeros((_NP, HID), jnp.float32)
    zeros8 = jnp.zeros((_NP, _OBW), jnp.float32)

    E8P = _EPAD // _PK
    EB8 = 2000
    grid = (E8 // EB8,)
    grid2 = (E8P // 2048,)
    ea_p = edge_attr.reshape(E8, 14 * _PK)
    wspecs1 = [_full((W256, W256))] + [_full(w.shape) for w in ein] + \
              [_full(C1.shape)] + [_full(w.shape) for w in etail1]
    e1p, ef1p = pl.pallas_call(
        _edge_stage1_body,
        grid=grid,
        in_specs=[pl.BlockSpec((EB8, 14 * _PK), lambda i: (i, 0))] + wspecs1,
        out_specs=[pl.BlockSpec((EB8, W256), lambda i: (i, 0))] * 2,
        out_shape=[jax.ShapeDtypeStruct((E8P, W256), jnp.float32)] * 2,
    )(ea_p, Mavg, *ein, C1, *etail1)

    NCH = _EPAD // _CHUNK
    srcp, dstp = pl.pallas_call(
        _pad_idx_body,
        grid=(NCH // 128,),
        in_specs=[pl.BlockSpec((128, _CHUNK), lambda i: (i, 0))] * 2,
        out_specs=[pl.BlockSpec((128, _CHUNK), lambda i: (i, 0))] * 2,
        out_shape=[jax.ShapeDtypeStruct((NCH, _CHUNK), jnp.int32)] * 2,
    )(src.reshape(_ECH, _CHUNK), dst.reshape(_ECH, _CHUNK))
    aggr1 = _sc_segsum(e1p.reshape(NCH, _CHUNK, HID), dstp, zeros32)

    a1p = aggr1.reshape(_NC, _NP8, W256)
    wspecsn1 = [_full((W256, W256)), _full(Q1.shape)] + \
               [_full(w.shape) for w in ntail1] + [_full(D2.shape)]
    nf1p, nf1d = pl.pallas_call(
        _node_stage1_body,
        grid=(1,),
        in_specs=[pl.BlockSpec((1, _NP8, W256), lambda i: (0, 0, 0)),
                  pl.BlockSpec((1, _NP8, W256), lambda i: (1, 0, 0))] + wspecsn1,
        out_specs=[pl.BlockSpec((_NP8, W256), lambda i: (0, 0))] * 2,
        out_shape=[jax.ShapeDtypeStruct((_NP8, W256), jnp.float32)] * 2,
    )(a1p, a1p, Mavg, Q1, *ntail1, D2)

    td = _sc_gatherdiff(nf1d.reshape(_NP, HID), srcp, dstp)

    wspecs2 = [_full((W256, W256)), _full(C2.shape)] + \
              [_full(w.shape) for w in etail2]
    e2p = pl.pallas_call(
        _edge_stage2_body,
        grid=grid2,
        in_specs=[pl.BlockSpec((2048, W256), lambda i: (i, 0))] * 2 + wspecs2,
        out_specs=pl.BlockSpec((2048, W256), lambda i: (i, 0)),
        out_shape=jax.ShapeDtypeStruct((E8P, W256), jnp.float32),
    )(td.reshape(E8P, W256), ef1p, Mavg, C2, *etail2)

    aggr2 = _sc_segsum(e2p.reshape(NCH, _CHUNK, HID), dstp, zeros32)

    VB = 8192
    vgrid = (_E3P // VB,)
    pad3 = _E3P - E3
    inv_p = jnp.pad(inv_edge_distance3, ((0, pad3), (0, 0)))
    app_p = jnp.pad(approach_speed, ((0, pad3), (0, 0)))
    dir_p = jnp.pad(edge_direction3, ((0, pad3), (0, 0)))
    idx3_p = jnp.pad(edge_index3[0], ((0, pad3),))
    vals8 = pl.pallas_call(
        _vals3_body,
        grid=vgrid,
        in_specs=[pl.BlockSpec((VB, 1), lambda i: (i, 0)),
                  pl.BlockSpec((VB, 1), lambda i: (i, 0)),
                  pl.BlockSpec((VB, 2), lambda i: (i, 0)),
                  _full((1, 8))],
        out_specs=pl.BlockSpec((VB, _OBW), lambda i: (i, 0)),
        out_shape=jax.ShapeDtypeStruct((_E3P, _OBW), jnp.float32),
    )(inv_p, app_p, dir_p, svec)
    ob_part = _sc_obscatter(vals8.reshape(_E3P // _CHUNK, _CHUNK, _OBW),
                            idx3_p.reshape(_E3P // _CHUNK, _CHUNK), zeros8)

    a2p = aggr2.reshape(_NC, _NP8, W256)
    wspecsn2 = ([_full((W256, W256)), _full(P2.shape), _full(Q2.shape)] +
                [_full(w.shape) for w in ntail2] +
                [_full(oW1.shape), _full(ob1w.shape), _full(oW2.shape)])
    swarm = pl.pallas_call(
        _node_stage2_body,
        grid=(1,),
        in_specs=[pl.BlockSpec((_NP8, W256), lambda i: (0, 0)),
                  pl.BlockSpec((1, _NP8, W256), lambda i: (0, 0, 0)),
                  pl.BlockSpec((1, _NP8, W256), lambda i: (1, 0, 0))] + wspecsn2,
        out_specs=pl.BlockSpec((_NP8, 2 * _PK), lambda i: (0, 0)),
        out_shape=jax.ShapeDtypeStruct((_NP8, 2 * _PK), jnp.float32),
    )(nf1p, a2p, a2p, Mavg, P2, Q2, *ntail2, oW1, ob1w, oW2)

    NB = 2000
    ngrid = (N // NB,)
    out = pl.pallas_call(
        _assemble_body,
        grid=ngrid,
        in_specs=[pl.BlockSpec((NB, 2), lambda i: (i, 0)),
                  pl.BlockSpec((1, NB, _OBW), lambda i: (0, i, 0)),
                  pl.BlockSpec((1, NB, _OBW), lambda i: (1, i, 0))] +
                 [pl.BlockSpec((NB, 2), lambda i: (i, 0))] * 3 +
                 [_full((1, 8))],
        out_specs=pl.BlockSpec((NB, 2), lambda i: (i, 0)),
        out_shape=jax.ShapeDtypeStruct((N, 2), jnp.float32),
    )(swarm.reshape(_NP, 2), ob_part, ob_part, recent_velocity,
      norm_inv_distance_to_boundary, norm_distance_to_boundary, svec)

    return out
